# Initial kernel scaffold; baseline (speedup 1.0000x reference)
import jax
import jax.numpy as jnp
from jax.experimental import pallas as pl


def kernel(x, edge_attr, params, edge_index, batch):
    raise NotImplementedError("write your pallas kernel here")


# speedup vs baseline: 1.1887x; 1.1887x over previous
import functools

import numpy as np
import jax
import jax.numpy as jnp
from jax import lax
from jax.experimental import pallas as pl
from jax.experimental.pallas import tpu as pltpu, tpu_sc as plsc

N = 50000
NP = 51200
E = 800000
EP = 802816
D = 64
NG = 64
HID = 128
HALF = NP // 2
CHUNK = 512
GRP = CHUNK // 16
PAD_NODE = 50008
NCORES = 2
NSUB = 16
DIM_ACT = 51
NPASS = 4
QUART = NP // (NCORES * NPASS)

_s3 = float(np.sqrt(3.0))
_s5 = float(np.sqrt(5.0))
_s15 = float(np.sqrt(15.0))


def _mk_big(W0, W1, W2, m_in, s0, s1, s2):
    m0, m1, m2 = m_in
    o0, o1, o2 = W0.shape[1], W1.shape[1], W2.shape[1]
    Wb = jnp.zeros((D, D), jnp.float32)
    Wb = Wb.at[:m0, :o0].set(W0 * s0)
    r, c = m0, o0
    Wb = Wb.at[r:r + 3 * m1, c:c + 3 * o1].set(jnp.kron(W1, jnp.eye(3)) * s1)
    r, c = r + 3 * m1, c + 3 * o1
    Wb = Wb.at[r:r + 5 * m2, c:c + 5 * o2].set(jnp.kron(W2, jnp.eye(5)) * s2)
    return Wb


def _layer_bigs(p, m_in):
    i0, i1, i2 = m_in
    sq = (1.0 / np.sqrt(i0), 1.0 / np.sqrt(i1), 1.0 / np.sqrt(i2))
    Wq = _mk_big(p['Wq0'], p['Wq1'], p['Wq2'], m_in, *sq)
    Wk = _mk_big(p['Wk0'], p['Wk1'], p['Wk2'], m_in, *sq)
    Wv = _mk_big(p['Wv0'], p['Wv1'], p['Wv2'], m_in, *sq)
    m_out = (11, 5, 5)
    sd = (1.0 / (11.0 * _s3),
          1.0 / (np.sqrt(25.0 * 3.0) * _s3),
          1.0 / (np.sqrt(25.0 * 5.0) * _s3))
    Wd = _mk_big(p['Wd0'][0], p['Wd1'][0], p['Wd2'][0], m_out, *sd)
    return Wq, Wk, Wv, Wd


def _group_mat():
    G = np.zeros((D, D), np.float32)
    for j in range(11):
        G[j, j] = 1.0
    for o in range(5):
        b = 11 + 3 * o
        G[b:b + 3, b:b + 3] = 1.0
    for o in range(5):
        b = 26 + 5 * o
        G[b:b + 5, b:b + 5] = 1.0
    for j in range(DIM_ACT, D):
        G[j, j] = 1.0
    return jnp.asarray(G)


_RB = 1024

_HP = lax.Precision.HIGHEST


def _dot(a, b):
    return lax.dot_general(a, b, (((1,), (0,)), ((), ())),
                           precision=_HP, preferred_element_type=jnp.float32)


def _norm_act(f, G):
    n2 = _dot(f * f, G)
    nrm = jnp.sqrt(n2 + 1e-10)
    return f * (jax.nn.sigmoid(nrm) / (nrm + 1e-5))


def _qkv_from_f(f, Wq, Wk, Wv, Wd, G):
    q = _norm_act(_dot(f, Wq), G)
    qt = _dot(q, Wd)
    k = _norm_act(_dot(f, Wk), G)
    v = _norm_act(_dot(f, Wv), G)
    return qt, k, v


def _embed_body(x_ref, ea_ref, We_ref, be_ref, Wq_ref, Wk_ref, Wv_ref,
                Wd_ref, G_ref, qt_ref, k_ref, v_ref):
    x = x_ref[...]
    ea = ea_ref[...]
    h = _dot(x, We_ref[...]) + be_ref[...]
    r = jnp.sqrt(jnp.sum(ea * ea, axis=1, keepdims=True) + 1e-12)
    u = ea / r
    ux, uy, uz = u[:, 0:1], u[:, 1:2], u[:, 2:3]
    sh = jnp.concatenate([
        jnp.ones_like(ux), _s3 * ux, _s3 * uy, _s3 * uz,
        _s15 * ux * uy, _s15 * uy * uz, (_s5 / 2.0) * (3.0 * uz * uz - 1.0),
        _s15 * ux * uz, (_s15 / 2.0) * (ux * ux - uy * uy)], axis=1)
    f = jnp.concatenate(
        [h, sh, jnp.zeros((x.shape[0], D - 19), jnp.float32)], axis=1)
    qt, k, v = _qkv_from_f(f, Wq_ref[...], Wk_ref[...], Wv_ref[...],
                           Wd_ref[...], G_ref[...])
    qt_ref[...] = qt
    k_ref[...] = k
    v_ref[...] = v


def _call_embed(xp, eap, We, be, Wq, Wk, Wv, Wd, G):
    row = pl.BlockSpec((_RB, D), lambda i: (i, 0))
    full = pl.BlockSpec(None, lambda i: (0, 0))
    out = jax.ShapeDtypeStruct((NP, D), jnp.float32)
    return pl.pallas_call(
        _embed_body,
        grid=(NP // _RB,),
        in_specs=[pl.BlockSpec((_RB, 10), lambda i: (i, 0)),
                  pl.BlockSpec((_RB, 3), lambda i: (i, 0)),
                  full, full, full, full, full, full, full],
        out_specs=[row, row, row],
        out_shape=[out, out, out],
    )(xp, eap, We, be, Wq, Wk, Wv, Wd, G)


def _dense_body(has_prev, *refs):
    if has_prev:
        (agg_ref, vp_ref, fp_ref, Wq_ref, Wk_ref, Wv_ref, Wd_ref, G_ref,
         f_ref, qt_ref, k_ref, v_ref) = refs
        f = agg_ref[...] + vp_ref[...] + fp_ref[...]
    else:
        (agg_ref, vp_ref, Wq_ref, Wk_ref, Wv_ref, Wd_ref, G_ref,
         f_ref, qt_ref, k_ref, v_ref) = refs
        f = agg_ref[...] + vp_ref[...]
    qt, k, v = _qkv_from_f(f, Wq_ref[...], Wk_ref[...], Wv_ref[...],
                           Wd_ref[...], G_ref[...])
    f_ref[...] = f
    qt_ref[...] = qt
    k_ref[...] = k
    v_ref[...] = v


def _call_dense(agg, vprev, fprev, Wq, Wk, Wv, Wd, G):
    has_prev = fprev is not None
    row = pl.BlockSpec((_RB, D), lambda i: (i, 0))
    full = pl.BlockSpec(None, lambda i: (0, 0))
    out = jax.ShapeDtypeStruct((NP, D), jnp.float32)
    args = [agg, vprev] + ([fprev] if has_prev else []) + [Wq, Wk, Wv, Wd, G]
    n_row = 3 if has_prev else 2
    return pl.pallas_call(
        functools.partial(_dense_body, has_prev),
        grid=(NP // _RB,),
        in_specs=[row] * n_row + [full] * 5,
        out_specs=[row, row, row, row],
        out_shape=[out, out, out, out],
    )(*args)


def _head_body(agg_ref, vp_ref, fp_ref, degp_ref, batch_ref, Wol_ref,
               W1_ref, b1_ref, W2_ref, b2_ref, Wo_ref, bo_ref,
               out_ref, acc_s, acc_c):
    i = pl.program_id(0)

    @pl.when(i == 0)
    def _():
        acc_s[...] = jnp.zeros_like(acc_s)
        acc_c[...] = jnp.zeros_like(acc_c)

    f = agg_ref[...] + vp_ref[...] + fp_ref[...]
    s = _dot(f, Wol_ref[...])
    nrm = jnp.sqrt(jnp.sum(s * s, axis=1, keepdims=True))
    s = s / jnp.maximum(nrm, 1e-12)
    s = jax.nn.relu(_dot(s, W1_ref[...]) + b1_ref[...])
    s = jax.nn.relu(_dot(s, W2_ref[...]) + b2_ref[...])
    deg = (degp_ref[0, :] + degp_ref[1, :]).reshape(-1, 1)
    gids = lax.broadcasted_iota(jnp.int32, (1, NG), 1)
    onehot = (batch_ref[...] == gids).astype(jnp.float32)
    ws = s * deg
    acc_s[...] += lax.dot_general(onehot, ws, (((0,), (0,)), ((), ())),
                                  precision=_HP,
                                  preferred_element_type=jnp.float32)
    acc_c[...] += lax.dot_general(
        onehot, jnp.broadcast_to(deg, ws.shape), (((0,), (0,)), ((), ())),
        precision=_HP, preferred_element_type=jnp.float32)

    @pl.when(i == pl.num_programs(0) - 1)
    def _():
        m = acc_s[...] / jnp.maximum(acc_c[...], 1.0)
        logits = _dot(m, Wo_ref[...]) + bo_ref[...]
        mx = jnp.max(logits, axis=1, keepdims=True)
        e = jnp.exp(logits - mx)
        out_ref[...] = e / jnp.sum(e, axis=1, keepdims=True)


def _call_head(agg, vprev, fprev, degp, batch2d, Wol, W1, b1, W2, b2, Wo, bo):
    row = pl.BlockSpec((_RB, D), lambda i: (i, 0))
    full = pl.BlockSpec(None, lambda i: (0, 0))
    return pl.pallas_call(
        _head_body,
        grid=(NP // _RB,),
        in_specs=[row, row, row,
                  pl.BlockSpec((2, _RB), lambda i: (0, i)),
                  pl.BlockSpec((_RB, 1), lambda i: (i, 0)),
                  full, full, full, full, full, full, full],
        out_specs=pl.BlockSpec(None, lambda i: (0, 0)),
        out_shape=jax.ShapeDtypeStruct((NG, 9), jnp.float32),
        scratch_shapes=[pltpu.VMEM((NG, HID), jnp.float32),
                        pltpu.VMEM((NG, HID), jnp.float32)],
    )(agg, vprev, fprev, degp, batch2d, Wol, W1, b1, W2, b2, Wo, bo)


_MESH = dict(core_axis_name="c", subcore_axis_name="s",
             num_cores=NCORES, num_subcores=NSUB)


def _lane():
    return lax.broadcasted_iota(jnp.int32, (16,), 0)


def _scores_body(with_deg, *refs):
    if with_deg:
        (qt_hbm, k_hbm, dst_hbm, src_hbm, ex_hbm, zp_hbm, degp_hbm,
         dst_v, src_v, qrows, krows, ex_v, stage_v, ones_v,
         z_sp, deg_sp, sem1, sem2) = refs
    else:
        (qt_hbm, k_hbm, dst_hbm, src_hbm, ex_hbm, zp_hbm,
         dst_v, src_v, qrows, krows, ex_v, stage_v,
         z_sp, sem1, sem2) = refs
    c = lax.axis_index("c")
    s = lax.axis_index("s")
    wid = s * NCORES + c
    zseg = NP // NSUB
    sz = pl.multiple_of(s * zseg, zseg)
    lane = _lane()

    def zero_loop(i, carry):
        stage_v[pl.ds(pl.multiple_of(i * 16, 16), 16)] = jnp.zeros(
            (16,), jnp.float32)
        return carry
    lax.fori_loop(0, zseg // 16, zero_loop, 0)
    pltpu.sync_copy(stage_v, z_sp.at[pl.ds(sz, zseg)])
    if with_deg:
        pltpu.sync_copy(stage_v, deg_sp.at[pl.ds(sz, zseg)])
    plsc.subcore_barrier()

    nchunk = EP // 32 // CHUNK
    base0 = wid * (EP // 32)

    def chunk_loop(ci, carry):
        base = pl.multiple_of(base0 + ci * CHUNK, CHUNK)
        pltpu.sync_copy(dst_hbm.at[pl.ds(base, CHUNK)], dst_v)
        pltpu.sync_copy(src_hbm.at[pl.ds(base, CHUNK)], src_v)
        cp1 = pltpu.async_copy(qt_hbm.at[dst_v], qrows, sem1)
        cp2 = pltpu.async_copy(k_hbm.at[src_v], krows, sem2)
        cp1.wait()
        cp2.wait()

        def grp_loop(g, carry2):
            o = pl.multiple_of(g * 16, 16)
            rows = o + lane
            acc = jnp.zeros((16,), jnp.float32)
            for j in range(DIM_ACT):
                colj = jnp.full((16,), j, jnp.int32)
                acc = acc + (plsc.load_gather(qrows, [rows, colj]) *
                             plsc.load_gather(krows, [rows, colj]))
            ex_v[pl.ds(o, 16)] = jnp.exp(acc)
            if with_deg:
                d16 = dst_v[pl.ds(o, 16)]
                ones_v[pl.ds(o, 16)] = jnp.where(
                    d16 < N, jnp.float32(1.0), jnp.float32(0.0))
            return carry2
        lax.fori_loop(0, GRP, grp_loop, 0)

        pltpu.sync_copy(ex_v, ex_hbm.at[pl.ds(base, CHUNK)])
        pltpu.sync_copy(ex_v, z_sp.at[dst_v], add=True)
        if with_deg:
            pltpu.sync_copy(ones_v, deg_sp.at[dst_v], add=True)
        return carry
    lax.fori_loop(0, nchunk, chunk_loop, 0)
    plsc.subcore_barrier()

    pltpu.sync_copy(z_sp.at[pl.ds(sz, zseg)],
                    zp_hbm.at[c, pl.ds(sz, zseg)])
    if with_deg:
        pltpu.sync_copy(deg_sp.at[pl.ds(sz, zseg)],
                        degp_hbm.at[c, pl.ds(sz, zseg)])


def _make_scores(with_deg):
    out_type = [jax.ShapeDtypeStruct((EP,), jnp.float32),
                jax.ShapeDtypeStruct((NCORES, NP), jnp.float32)]
    scratch = [pltpu.VMEM((CHUNK,), jnp.int32),
               pltpu.VMEM((CHUNK,), jnp.int32),
               pltpu.VMEM((CHUNK, D), jnp.float32),
               pltpu.VMEM((CHUNK, D), jnp.float32),
               pltpu.VMEM((CHUNK,), jnp.float32),
               pltpu.VMEM((NP // NSUB,), jnp.float32)]
    if with_deg:
        out_type.append(jax.ShapeDtypeStruct((NCORES, NP), jnp.float32))
        scratch.append(pltpu.VMEM((CHUNK,), jnp.float32))
    scratch.append(pltpu.VMEM_SHARED((NP,), jnp.float32))
    if with_deg:
        scratch.append(pltpu.VMEM_SHARED((NP,), jnp.float32))
    scratch += [pltpu.SemaphoreType.DMA, pltpu.SemaphoreType.DMA]
    return pl.kernel(
        functools.partial(_scores_body, with_deg),
        out_type=tuple(out_type),
        mesh=plsc.VectorSubcoreMesh(**_MESH),
        scratch_types=scratch,
        compiler_params=pltpu.CompilerParams(needs_layout_passes=False, use_tc_tiling_on_sc=False),
    )


@functools.cache
def _scores_deg():
    return _make_scores(True)


@functools.cache
def _scores():
    return _make_scores(False)


def _rsqrt_pos(x):
    i = plsc.bitcast(x, jnp.int32)
    i = jnp.int32(0x5F3759DF) - lax.shift_right_logical(i, 1)
    y = plsc.bitcast(i, jnp.float32)
    for _ in range(3):
        y = y * (1.5 - 0.5 * x * y * y)
    return y


def _agg_body(v_hbm, ex_hbm, zp_hbm, dst_hbm, src_hbm, agg_hbm,
              dst_v, src_v, ex_v, idx_v, vrows, zrows, zloc, ztmp, fout_sp,
              sem1):
    c = lax.axis_index("c")
    s = lax.axis_index("s")
    lane = _lane()
    rows_per_tile = QUART // NSUB
    rbase = pl.multiple_of(s * rows_per_tile, rows_per_tile)
    nchunk = EP // NSUB // CHUNK
    base0 = s * (EP // NSUB)

    def vz(i, carry):
        for cj in range(D // 16):
            zrows[i, pl.ds(cj * 16, 16)] = jnp.zeros((16,), jnp.float32)
        return carry
    lax.fori_loop(0, CHUNK, vz, 0)

    for pi in range(NPASS):
        lo = pl.multiple_of((pi * NCORES + c) * QUART, QUART)

        pltpu.sync_copy(zp_hbm.at[0, pl.ds(lo, QUART)], zloc)
        pltpu.sync_copy(zp_hbm.at[1, pl.ds(lo, QUART)], ztmp)

        def zadd(i, carry):
            o = pl.multiple_of(i * 16, 16)
            zloc[pl.ds(o, 16)] = zloc[pl.ds(o, 16)] + ztmp[pl.ds(o, 16)]
            return carry
        lax.fori_loop(0, QUART // 16, zadd, 0)

        done = 0
        while done < rows_per_tile:
            step = min(CHUNK, rows_per_tile - done)
            pltpu.sync_copy(zrows.at[pl.ds(0, step)],
                            fout_sp.at[pl.ds(rbase + done, step)])
            done += step
        plsc.subcore_barrier()

        def chunk_loop(ci, carry):
            base = pl.multiple_of(base0 + ci * CHUNK, CHUNK)
            pltpu.sync_copy(dst_hbm.at[pl.ds(base, CHUNK)], dst_v)
            pltpu.sync_copy(src_hbm.at[pl.ds(base, CHUNK)], src_v)
            pltpu.sync_copy(ex_hbm.at[pl.ds(base, CHUNK)], ex_v)
            pltpu.async_copy(v_hbm.at[src_v], vrows, sem1).wait()

            def grp_loop(g, carry2):
                o = pl.multiple_of(g * 16, 16)
                rows = o + lane
                d16 = dst_v[pl.ds(o, 16)]
                loc = d16 - lo
                owned = (loc >= 0) & (loc < QUART)
                locc = jnp.clip(loc, 0, QUART - 1)
                idx_v[pl.ds(o, 16)] = locc
                zv = plsc.load_gather(zloc, [locc])
                exv = ex_v[pl.ds(o, 16)]
                ratio = jnp.where(
                    zv > 0.0, exv / jnp.where(zv > 0.0, zv, 1.0), 0.0)
                w = jnp.where(ratio > 0.0, ratio * _rsqrt_pos(ratio), 0.0)
                w = jnp.where(owned, w, 0.0)
                for j in range(DIM_ACT):
                    colj = jnp.full((16,), j, jnp.int32)
                    vc = plsc.load_gather(vrows, [rows, colj])
                    plsc.store_scatter(vrows, [rows, colj], vc * w)
                return carry2
            lax.fori_loop(0, GRP, grp_loop, 0)

            pltpu.sync_copy(vrows, fout_sp.at[idx_v], add=True)
            return carry
        lax.fori_loop(0, nchunk, chunk_loop, 0)
        plsc.subcore_barrier()

        pltpu.sync_copy(fout_sp.at[pl.ds(rbase, rows_per_tile)],
                        agg_hbm.at[pl.ds(lo + rbase, rows_per_tile)])


@functools.cache
def _agg():
    return pl.kernel(
        _agg_body,
        out_type=jax.ShapeDtypeStruct((NP, D), jnp.float32),
        mesh=plsc.VectorSubcoreMesh(**_MESH),
        scratch_types=[pltpu.VMEM((CHUNK,), jnp.int32),
                       pltpu.VMEM((CHUNK,), jnp.int32),
                       pltpu.VMEM((CHUNK,), jnp.float32),
                       pltpu.VMEM((CHUNK,), jnp.int32),
                       pltpu.VMEM((CHUNK, D), jnp.float32),
                       pltpu.VMEM((CHUNK, D), jnp.float32),
                       pltpu.VMEM((QUART,), jnp.float32),
                       pltpu.VMEM((QUART,), jnp.float32),
                       pltpu.VMEM_SHARED((QUART, D), jnp.float32),
                       pltpu.SemaphoreType.DMA],
        compiler_params=pltpu.CompilerParams(
            needs_layout_passes=False, use_tc_tiling_on_sc=False),
    )


def kernel(x, edge_attr, params, edge_index, batch):
    p = params
    src = edge_index[0].astype(jnp.int32)
    dst = edge_index[1].astype(jnp.int32)

    xp = jnp.pad(x, ((0, NP - N), (0, 0)))
    eap = jnp.pad(edge_attr, ((0, NP - N), (0, 0)))
    padE = jnp.full((EP - E,), PAD_NODE, jnp.int32)
    dst_p = jnp.concatenate([dst, padE])
    src_p = jnp.concatenate([src, padE])
    batch2d = jnp.pad(batch.astype(jnp.int32), (0, NP - N)).reshape(NP, 1)

    G = _group_mat()
    Wq0, Wk0, Wv0, Wd0 = _layer_bigs(p['et'], (11, 1, 1))
    Wq1, Wk1, Wv1, Wd1 = _layer_bigs(p['m_et'][0], (11, 5, 5))
    Wq2, Wk2, Wv2, Wd2 = _layer_bigs(p['m_et'][1], (11, 5, 5))
    be = p['b_embd'].reshape(1, 10)
    Wol = jnp.zeros((D, HID), jnp.float32).at[:11, :].set(
        p['W_ol'] / np.sqrt(11.0))
    (W1h, b1h), (W2h, b2h) = p['lin']
    b1h = b1h.reshape(1, HID)
    b2h = b2h.reshape(1, HID)
    bo = p['b_out'].reshape(1, 9)

    qt, k, v = _call_embed(xp, eap, p['W_embd'], be, Wq0, Wk0, Wv0, Wd0, G)
    ex, zp, degp = _scores_deg()(qt, k, dst_p, src_p)
    agg = _agg()(v, ex, zp, dst_p, src_p)

    f1, qt, k, v1 = _call_dense(agg, v, None, Wq1, Wk1, Wv1, Wd1, G)
    ex, zp = _scores()(qt, k, dst_p, src_p)
    agg = _agg()(v1, ex, zp, dst_p, src_p)

    f2, qt, k, v2 = _call_dense(agg, v1, f1, Wq2, Wk2, Wv2, Wd2, G)
    ex, zp = _scores()(qt, k, dst_p, src_p)
    agg = _agg()(v2, ex, zp, dst_p, src_p)

    return _call_head(agg, v2, f2, degp, batch2d,
                      Wol, W1h, b1h, W2h, b2h, p['W_out'], bo)


# speedup vs baseline: 3.7087x; 3.1199x over previous
import functools

import numpy as np
import jax
import jax.numpy as jnp
from jax import lax
from jax.experimental import pallas as pl
from jax.experimental.pallas import tpu as pltpu, tpu_sc as plsc

N = 50000
NP = 51200
E = 800000
EP = 802816
D = 64
NG = 64
HID = 128
HALF = NP // 2
CHUNK = 512
GRP = CHUNK // 16
PAD_NODE = 50008
NCORES = 2
NSUB = 16
DIM_ACT = 51
NPASS = 4
QUART = NP // (NCORES * NPASS)
FIRE = 512

_s3 = float(np.sqrt(3.0))
_s5 = float(np.sqrt(5.0))
_s15 = float(np.sqrt(15.0))


def _mk_big(W0, W1, W2, m_in, s0, s1, s2):
    m0, m1, m2 = m_in
    o0, o1, o2 = W0.shape[1], W1.shape[1], W2.shape[1]
    Wb = jnp.zeros((D, D), jnp.float32)
    Wb = Wb.at[:m0, :o0].set(W0 * s0)
    r, c = m0, o0
    Wb = Wb.at[r:r + 3 * m1, c:c + 3 * o1].set(jnp.kron(W1, jnp.eye(3)) * s1)
    r, c = r + 3 * m1, c + 3 * o1
    Wb = Wb.at[r:r + 5 * m2, c:c + 5 * o2].set(jnp.kron(W2, jnp.eye(5)) * s2)
    return Wb


def _layer_bigs(p, m_in):
    i0, i1, i2 = m_in
    sq = (1.0 / np.sqrt(i0), 1.0 / np.sqrt(i1), 1.0 / np.sqrt(i2))
    Wq = _mk_big(p['Wq0'], p['Wq1'], p['Wq2'], m_in, *sq)
    Wk = _mk_big(p['Wk0'], p['Wk1'], p['Wk2'], m_in, *sq)
    Wv = _mk_big(p['Wv0'], p['Wv1'], p['Wv2'], m_in, *sq)
    m_out = (11, 5, 5)
    sd = (1.0 / (11.0 * _s3),
          1.0 / (np.sqrt(25.0 * 3.0) * _s3),
          1.0 / (np.sqrt(25.0 * 5.0) * _s3))
    Wd = _mk_big(p['Wd0'][0], p['Wd1'][0], p['Wd2'][0], m_out, *sd)
    return Wq, Wk, Wv, Wd


def _group_mat():
    G = np.zeros((D, D), np.float32)
    for j in range(11):
        G[j, j] = 1.0
    for o in range(5):
        b = 11 + 3 * o
        G[b:b + 3, b:b + 3] = 1.0
    for o in range(5):
        b = 26 + 5 * o
        G[b:b + 5, b:b + 5] = 1.0
    for j in range(DIM_ACT, D):
        G[j, j] = 1.0
    return jnp.asarray(G)


_RB = 1024

_HP = lax.Precision.HIGHEST


def _dot(a, b):
    return lax.dot_general(a, b, (((1,), (0,)), ((), ())),
                           precision=_HP, preferred_element_type=jnp.float32)


def _norm_act(f, G):
    n2 = _dot(f * f, G)
    nrm = jnp.sqrt(n2 + 1e-10)
    return f * (jax.nn.sigmoid(nrm) / (nrm + 1e-5))


def _qkv_from_f(f, Wq, Wk, Wv, Wd, G):
    q = _norm_act(_dot(f, Wq), G)
    qt = _dot(q, Wd)
    k = _norm_act(_dot(f, Wk), G)
    v = _norm_act(_dot(f, Wv), G)
    return qt, k, v


def _embed_body(x_ref, ea_ref, We_ref, be_ref, Wq_ref, Wk_ref, Wv_ref,
                Wd_ref, G_ref, qt_ref, k_ref, v_ref):
    x = x_ref[...]
    ea = ea_ref[...]
    h = _dot(x, We_ref[...]) + be_ref[...]
    r = jnp.sqrt(jnp.sum(ea * ea, axis=1, keepdims=True) + 1e-12)
    u = ea / r
    ux, uy, uz = u[:, 0:1], u[:, 1:2], u[:, 2:3]
    sh = jnp.concatenate([
        jnp.ones_like(ux), _s3 * ux, _s3 * uy, _s3 * uz,
        _s15 * ux * uy, _s15 * uy * uz, (_s5 / 2.0) * (3.0 * uz * uz - 1.0),
        _s15 * ux * uz, (_s15 / 2.0) * (ux * ux - uy * uy)], axis=1)
    f = jnp.concatenate(
        [h, sh, jnp.zeros((x.shape[0], D - 19), jnp.float32)], axis=1)
    qt, k, v = _qkv_from_f(f, Wq_ref[...], Wk_ref[...], Wv_ref[...],
                           Wd_ref[...], G_ref[...])
    qt_ref[...] = qt
    k_ref[...] = k
    v_ref[...] = v


def _call_embed(xp, eap, We, be, Wq, Wk, Wv, Wd, G):
    row = pl.BlockSpec((_RB, D), lambda i: (i, 0))
    full = pl.BlockSpec(None, lambda i: (0, 0))
    out = jax.ShapeDtypeStruct((NP, D), jnp.float32)
    return pl.pallas_call(
        _embed_body,
        grid=(NP // _RB,),
        in_specs=[pl.BlockSpec((_RB, 10), lambda i: (i, 0)),
                  pl.BlockSpec((_RB, 3), lambda i: (i, 0)),
                  full, full, full, full, full, full, full],
        out_specs=[row, row, row],
        out_shape=[out, out, out],
    )(xp, eap, We, be, Wq, Wk, Wv, Wd, G)


def _dense_body(has_prev, *refs):
    if has_prev:
        (agg_ref, vp_ref, fp_ref, Wq_ref, Wk_ref, Wv_ref, Wd_ref, G_ref,
         f_ref, qt_ref, k_ref, v_ref) = refs
        f = agg_ref[...] + vp_ref[...] + fp_ref[...]
    else:
        (agg_ref, vp_ref, Wq_ref, Wk_ref, Wv_ref, Wd_ref, G_ref,
         f_ref, qt_ref, k_ref, v_ref) = refs
        f = agg_ref[...] + vp_ref[...]
    qt, k, v = _qkv_from_f(f, Wq_ref[...], Wk_ref[...], Wv_ref[...],
                           Wd_ref[...], G_ref[...])
    f_ref[...] = f
    qt_ref[...] = qt
    k_ref[...] = k
    v_ref[...] = v


def _call_dense(agg, vprev, fprev, Wq, Wk, Wv, Wd, G):
    has_prev = fprev is not None
    row = pl.BlockSpec((_RB, D), lambda i: (i, 0))
    full = pl.BlockSpec(None, lambda i: (0, 0))
    out = jax.ShapeDtypeStruct((NP, D), jnp.float32)
    args = [agg, vprev] + ([fprev] if has_prev else []) + [Wq, Wk, Wv, Wd, G]
    n_row = 3 if has_prev else 2
    return pl.pallas_call(
        functools.partial(_dense_body, has_prev),
        grid=(NP // _RB,),
        in_specs=[row] * n_row + [full] * 5,
        out_specs=[row, row, row, row],
        out_shape=[out, out, out, out],
    )(*args)


def _head_body(agg_ref, vp_ref, fp_ref, degp_ref, batch_ref, Wol_ref,
               W1_ref, b1_ref, W2_ref, b2_ref, Wo_ref, bo_ref,
               out_ref, acc_s, acc_c):
    i = pl.program_id(0)

    @pl.when(i == 0)
    def _():
        acc_s[...] = jnp.zeros_like(acc_s)
        acc_c[...] = jnp.zeros_like(acc_c)

    f = agg_ref[...] + vp_ref[...] + fp_ref[...]
    s = _dot(f, Wol_ref[...])
    nrm = jnp.sqrt(jnp.sum(s * s, axis=1, keepdims=True))
    s = s / jnp.maximum(nrm, 1e-12)
    s = jax.nn.relu(_dot(s, W1_ref[...]) + b1_ref[...])
    s = jax.nn.relu(_dot(s, W2_ref[...]) + b2_ref[...])
    deg = (degp_ref[0, :] + degp_ref[1, :]).reshape(-1, 1)
    gids = lax.broadcasted_iota(jnp.int32, (1, NG), 1)
    onehot = (batch_ref[...] == gids).astype(jnp.float32)
    ws = s * deg
    acc_s[...] += lax.dot_general(onehot, ws, (((0,), (0,)), ((), ())),
                                  precision=_HP,
                                  preferred_element_type=jnp.float32)
    acc_c[...] += lax.dot_general(
        onehot, jnp.broadcast_to(deg, ws.shape), (((0,), (0,)), ((), ())),
        precision=_HP, preferred_element_type=jnp.float32)

    @pl.when(i == pl.num_programs(0) - 1)
    def _():
        m = acc_s[...] / jnp.maximum(acc_c[...], 1.0)
        logits = _dot(m, Wo_ref[...]) + bo_ref[...]
        mx = jnp.max(logits, axis=1, keepdims=True)
        e = jnp.exp(logits - mx)
        out_ref[...] = e / jnp.sum(e, axis=1, keepdims=True)


def _call_head(agg, vprev, fprev, degp, batch2d, Wol, W1, b1, W2, b2, Wo, bo):
    row = pl.BlockSpec((_RB, D), lambda i: (i, 0))
    full = pl.BlockSpec(None, lambda i: (0, 0))
    return pl.pallas_call(
        _head_body,
        grid=(NP // _RB,),
        in_specs=[row, row, row,
                  pl.BlockSpec((2, _RB), lambda i: (0, i)),
                  pl.BlockSpec((_RB, 1), lambda i: (i, 0)),
                  full, full, full, full, full, full, full],
        out_specs=pl.BlockSpec(None, lambda i: (0, 0)),
        out_shape=jax.ShapeDtypeStruct((NG, 9), jnp.float32),
        scratch_shapes=[pltpu.VMEM((NG, HID), jnp.float32),
                        pltpu.VMEM((NG, HID), jnp.float32)],
    )(agg, vprev, fprev, degp, batch2d, Wol, W1, b1, W2, b2, Wo, bo)


_MESH = dict(core_axis_name="c", subcore_axis_name="s",
             num_cores=NCORES, num_subcores=NSUB)


def _lane():
    return lax.broadcasted_iota(jnp.int32, (16,), 0)


def _scores_body(with_deg, *refs):
    if with_deg:
        (qt_hbm, k_hbm, dst_hbm, src_hbm, ex_hbm, zp_hbm, degp_hbm,
         dst_v, src_v, qrows, krows, ex_v, stage_v, ones_v,
         z_sp, deg_sp, sem1, sem2) = refs
    else:
        (qt_hbm, k_hbm, dst_hbm, src_hbm, ex_hbm, zp_hbm,
         dst_v, src_v, qrows, krows, ex_v, stage_v,
         z_sp, sem1, sem2) = refs
    c = lax.axis_index("c")
    s = lax.axis_index("s")
    wid = s * NCORES + c
    zseg = NP // NSUB
    sz = pl.multiple_of(s * zseg, zseg)
    lane = _lane()

    def zero_loop(i, carry):
        stage_v[pl.ds(pl.multiple_of(i * 16, 16), 16)] = jnp.zeros(
            (16,), jnp.float32)
        return carry
    lax.fori_loop(0, zseg // 16, zero_loop, 0)
    pltpu.sync_copy(stage_v, z_sp.at[pl.ds(sz, zseg)])
    if with_deg:
        pltpu.sync_copy(stage_v, deg_sp.at[pl.ds(sz, zseg)])
    plsc.subcore_barrier()

    nchunk = EP // 32 // CHUNK
    base0 = wid * (EP // 32)

    def chunk_loop(ci, carry):
        base = pl.multiple_of(base0 + ci * CHUNK, CHUNK)
        pltpu.sync_copy(dst_hbm.at[pl.ds(base, CHUNK)], dst_v)
        pltpu.sync_copy(src_hbm.at[pl.ds(base, CHUNK)], src_v)
        cp1 = pltpu.async_copy(qt_hbm.at[dst_v], qrows, sem1)
        cp2 = pltpu.async_copy(k_hbm.at[src_v], krows, sem2)
        cp1.wait()
        cp2.wait()

        def grp_loop(g, carry2):
            o = pl.multiple_of(g * 16, 16)
            rows = o + lane
            acc = jnp.zeros((16,), jnp.float32)
            for j in range(DIM_ACT):
                colj = jnp.full((16,), j, jnp.int32)
                acc = acc + (plsc.load_gather(qrows, [rows, colj]) *
                             plsc.load_gather(krows, [rows, colj]))
            ex_v[pl.ds(o, 16)] = jnp.exp(acc)
            if with_deg:
                d16 = dst_v[pl.ds(o, 16)]
                ones_v[pl.ds(o, 16)] = jnp.where(
                    d16 < N, jnp.float32(1.0), jnp.float32(0.0))
            return carry2
        lax.fori_loop(0, GRP, grp_loop, 0)

        pltpu.sync_copy(ex_v, ex_hbm.at[pl.ds(base, CHUNK)])
        pltpu.sync_copy(ex_v, z_sp.at[dst_v], add=True)
        if with_deg:
            pltpu.sync_copy(ones_v, deg_sp.at[dst_v], add=True)
        return carry
    lax.fori_loop(0, nchunk, chunk_loop, 0)
    plsc.subcore_barrier()

    pltpu.sync_copy(z_sp.at[pl.ds(sz, zseg)],
                    zp_hbm.at[c, pl.ds(sz, zseg)])
    if with_deg:
        pltpu.sync_copy(deg_sp.at[pl.ds(sz, zseg)],
                        degp_hbm.at[c, pl.ds(sz, zseg)])


def _make_scores(with_deg):
    out_type = [jax.ShapeDtypeStruct((EP,), jnp.float32),
                jax.ShapeDtypeStruct((NCORES, NP), jnp.float32)]
    scratch = [pltpu.VMEM((CHUNK,), jnp.int32),
               pltpu.VMEM((CHUNK,), jnp.int32),
               pltpu.VMEM((CHUNK, D), jnp.float32),
               pltpu.VMEM((CHUNK, D), jnp.float32),
               pltpu.VMEM((CHUNK,), jnp.float32),
               pltpu.VMEM((NP // NSUB,), jnp.float32)]
    if with_deg:
        out_type.append(jax.ShapeDtypeStruct((NCORES, NP), jnp.float32))
        scratch.append(pltpu.VMEM((CHUNK,), jnp.float32))
    scratch.append(pltpu.VMEM_SHARED((NP,), jnp.float32))
    if with_deg:
        scratch.append(pltpu.VMEM_SHARED((NP,), jnp.float32))
    scratch += [pltpu.SemaphoreType.DMA, pltpu.SemaphoreType.DMA]
    return pl.kernel(
        functools.partial(_scores_body, with_deg),
        out_type=tuple(out_type),
        mesh=plsc.VectorSubcoreMesh(**_MESH),
        scratch_types=scratch,
        compiler_params=pltpu.CompilerParams(needs_layout_passes=False, use_tc_tiling_on_sc=False),
    )


@functools.cache
def _scores_deg():
    return _make_scores(True)


@functools.cache
def _scores():
    return _make_scores(False)


def _rsqrt_pos(x):
    i = plsc.bitcast(x, jnp.int32)
    i = jnp.int32(0x5F3759DF) - lax.shift_right_logical(i, 1)
    y = plsc.bitcast(i, jnp.float32)
    for _ in range(3):
        y = y * (1.5 - 0.5 * x * y * y)
    return y


def _agg_body(v_hbm, ex_hbm, zp_hbm, dst_hbm, src_hbm, agg_hbm,
              dst_v, src_v, ex_v, st_src, st_loc, st_w, vrows, zloc, ztmp,
              fout_sp, sem1):
    c = lax.axis_index("c")
    s = lax.axis_index("s")
    lane = _lane()
    lane_f = lane.astype(jnp.float32) * 0.0
    rows_per_tile = QUART // NSUB
    rbase = pl.multiple_of(s * rows_per_tile, rows_per_tile)
    nchunk = EP // NSUB // CHUNK
    base0 = s * (EP // NSUB)
    zero16f = jnp.zeros((16,), jnp.float32)
    zero16i = jnp.zeros((16,), jnp.int32)

    def fire():
        pltpu.async_copy(v_hbm.at[st_src], vrows, sem1).wait()

        def scale_loop(g, carry):
            o = pl.multiple_of(g * 16, 16)
            rows = o + lane
            w16 = st_w[pl.ds(o, 16)]
            for j in range(DIM_ACT):
                colj = jnp.full((16,), j, jnp.int32)
                vc = plsc.load_gather(vrows, [rows, colj])
                plsc.store_scatter(vrows, [rows, colj], vc * w16)
            return carry
        lax.fori_loop(0, FIRE // 16, scale_loop, 0)
        pltpu.sync_copy(vrows, fout_sp.at[st_loc], add=True)

    def pad_tail(cur):
        def pad_loop(g, carry):
            idx = g * 16 + lane
            m = (idx >= cur) & (idx < FIRE)
            plsc.store_scatter(st_src, [idx], zero16i, mask=m)
            plsc.store_scatter(st_loc, [idx], zero16i, mask=m)
            plsc.store_scatter(st_w, [idx], zero16f, mask=m)
            return carry
        lax.fori_loop(0, FIRE // 16, pad_loop, 0)

    for pi in range(NPASS):
        lo = pl.multiple_of((pi * NCORES + c) * QUART, QUART)

        pltpu.sync_copy(zp_hbm.at[0, pl.ds(lo, QUART)], zloc)
        pltpu.sync_copy(zp_hbm.at[1, pl.ds(lo, QUART)], ztmp)

        def zadd(i, carry):
            o = pl.multiple_of(i * 16, 16)
            zloc[pl.ds(o, 16)] = zloc[pl.ds(o, 16)] + ztmp[pl.ds(o, 16)]
            return carry
        lax.fori_loop(0, QUART // 16, zadd, 0)

        def vz(i, carry):
            for cj in range(D // 16):
                vrows[i, pl.ds(cj * 16, 16)] = zero16f
            return carry
        lax.fori_loop(0, FIRE, vz, 0)
        done = 0
        while done < rows_per_tile:
            step = min(FIRE, rows_per_tile - done)
            pltpu.sync_copy(vrows.at[pl.ds(0, step)],
                            fout_sp.at[pl.ds(rbase + done, step)])
            done += step
        plsc.subcore_barrier()

        def chunk_loop(ci, cur):
            base = pl.multiple_of(base0 + ci * CHUNK, CHUNK)
            pltpu.sync_copy(dst_hbm.at[pl.ds(base, CHUNK)], dst_v)
            pltpu.sync_copy(src_hbm.at[pl.ds(base, CHUNK)], src_v)
            pltpu.sync_copy(ex_hbm.at[pl.ds(base, CHUNK)], ex_v)

            def grp_loop(g, cur2):
                o = pl.multiple_of(g * 16, 16)
                d16 = dst_v[pl.ds(o, 16)]
                loc = d16 - lo
                owned = (loc >= 0) & (loc < QUART)
                locc = jnp.clip(loc, 0, QUART - 1)
                zv = plsc.load_gather(zloc, [locc])
                exv = ex_v[pl.ds(o, 16)]
                ratio = jnp.where(
                    zv > 0.0, exv / jnp.where(zv > 0.0, zv, 1.0), 0.0)
                w = jnp.where(ratio > 0.0, ratio * _rsqrt_pos(ratio), 0.0)
                ranks = plsc.cumsum(owned.astype(jnp.int32))
                n = jnp.sum(owned.astype(jnp.int32), axis=0)
                pos = jnp.clip(cur2 + ranks - 1, 0, FIRE - 1)
                plsc.store_scatter(st_src, [pos], src_v[pl.ds(o, 16)],
                                   mask=owned)
                plsc.store_scatter(st_loc, [pos], locc, mask=owned)
                plsc.store_scatter(st_w, [pos], w, mask=owned)
                cur3 = cur2 + n

                @pl.when(cur3 > FIRE - 16)
                def _():
                    pad_tail(cur3)
                    fire()
                return jnp.where(cur3 > FIRE - 16, 0, cur3)
            return lax.fori_loop(0, GRP, grp_loop, cur)
        cur = lax.fori_loop(0, nchunk, chunk_loop, jnp.int32(0))

        pad_tail(cur)
        fire()
        plsc.subcore_barrier()

        pltpu.sync_copy(fout_sp.at[pl.ds(rbase, rows_per_tile)],
                        agg_hbm.at[pl.ds(lo + rbase, rows_per_tile)])


@functools.cache
def _agg():
    return pl.kernel(
        _agg_body,
        out_type=jax.ShapeDtypeStruct((NP, D), jnp.float32),
        mesh=plsc.VectorSubcoreMesh(**_MESH),
        scratch_types=[pltpu.VMEM((CHUNK,), jnp.int32),
                       pltpu.VMEM((CHUNK,), jnp.int32),
                       pltpu.VMEM((CHUNK,), jnp.float32),
                       pltpu.VMEM((FIRE,), jnp.int32),
                       pltpu.VMEM((FIRE,), jnp.int32),
                       pltpu.VMEM((FIRE,), jnp.float32),
                       pltpu.VMEM((FIRE, D), jnp.float32),
                       pltpu.VMEM((QUART,), jnp.float32),
                       pltpu.VMEM((QUART,), jnp.float32),
                       pltpu.VMEM_SHARED((QUART, D), jnp.float32),
                       pltpu.SemaphoreType.DMA],
        compiler_params=pltpu.CompilerParams(
            needs_layout_passes=False, use_tc_tiling_on_sc=False),
    )


def kernel(x, edge_attr, params, edge_index, batch):
    p = params
    src = edge_index[0].astype(jnp.int32)
    dst = edge_index[1].astype(jnp.int32)

    xp = jnp.pad(x, ((0, NP - N), (0, 0)))
    eap = jnp.pad(edge_attr, ((0, NP - N), (0, 0)))
    padE = jnp.full((EP - E,), PAD_NODE, jnp.int32)
    dst_p = jnp.concatenate([dst, padE])
    src_p = jnp.concatenate([src, padE])
    batch2d = jnp.pad(batch.astype(jnp.int32), (0, NP - N)).reshape(NP, 1)

    G = _group_mat()
    Wq0, Wk0, Wv0, Wd0 = _layer_bigs(p['et'], (11, 1, 1))
    Wq1, Wk1, Wv1, Wd1 = _layer_bigs(p['m_et'][0], (11, 5, 5))
    Wq2, Wk2, Wv2, Wd2 = _layer_bigs(p['m_et'][1], (11, 5, 5))
    be = p['b_embd'].reshape(1, 10)
    Wol = jnp.zeros((D, HID), jnp.float32).at[:11, :].set(
        p['W_ol'] / np.sqrt(11.0))
    (W1h, b1h), (W2h, b2h) = p['lin']
    b1h = b1h.reshape(1, HID)
    b2h = b2h.reshape(1, HID)
    bo = p['b_out'].reshape(1, 9)

    qt, k, v = _call_embed(xp, eap, p['W_embd'], be, Wq0, Wk0, Wv0, Wd0, G)
    ex, zp, degp = _scores_deg()(qt, k, dst_p, src_p)
    agg = _agg()(v, ex, zp, dst_p, src_p)

    f1, qt, k, v1 = _call_dense(agg, v, None, Wq1, Wk1, Wv1, Wd1, G)
    ex, zp = _scores()(qt, k, dst_p, src_p)
    agg = _agg()(v1, ex, zp, dst_p, src_p)

    f2, qt, k, v2 = _call_dense(agg, v1, f1, Wq2, Wk2, Wv2, Wd2, G)
    ex, zp = _scores()(qt, k, dst_p, src_p)
    agg = _agg()(v2, ex, zp, dst_p, src_p)

    return _call_head(agg, v2, f2, degp, batch2d,
                      Wol, W1h, b1h, W2h, b2h, p['W_out'], bo)


# speedup vs baseline: 4.1127x; 1.1089x over previous
import functools

import numpy as np
import jax
import jax.numpy as jnp
from jax import lax
from jax.experimental import pallas as pl
from jax.experimental.pallas import tpu as pltpu, tpu_sc as plsc

N = 50000
NP = 51200
E = 800000
EP = 802816
D = 64
NG = 64
HID = 128
HALF = NP // 2
CHUNK = 512
GRP = CHUNK // 16
PAD_NODE = 50008
NCORES = 2
NSUB = 16
DIM_ACT = 51
NPASS = 4
QUART = NP // (NCORES * NPASS)
FIRE = 512
CHUNK_B = 3584
GRP_B = CHUNK_B // 16

_s3 = float(np.sqrt(3.0))
_s5 = float(np.sqrt(5.0))
_s15 = float(np.sqrt(15.0))


def _mk_big(W0, W1, W2, m_in, s0, s1, s2):
    m0, m1, m2 = m_in
    o0, o1, o2 = W0.shape[1], W1.shape[1], W2.shape[1]
    Wb = jnp.zeros((D, D), jnp.float32)
    Wb = Wb.at[:m0, :o0].set(W0 * s0)
    r, c = m0, o0
    Wb = Wb.at[r:r + 3 * m1, c:c + 3 * o1].set(jnp.kron(W1, jnp.eye(3)) * s1)
    r, c = r + 3 * m1, c + 3 * o1
    Wb = Wb.at[r:r + 5 * m2, c:c + 5 * o2].set(jnp.kron(W2, jnp.eye(5)) * s2)
    return Wb


def _layer_bigs(p, m_in):
    i0, i1, i2 = m_in
    sq = (1.0 / np.sqrt(i0), 1.0 / np.sqrt(i1), 1.0 / np.sqrt(i2))
    Wq = _mk_big(p['Wq0'], p['Wq1'], p['Wq2'], m_in, *sq)
    Wk = _mk_big(p['Wk0'], p['Wk1'], p['Wk2'], m_in, *sq)
    Wv = _mk_big(p['Wv0'], p['Wv1'], p['Wv2'], m_in, *sq)
    m_out = (11, 5, 5)
    sd = (1.0 / (11.0 * _s3),
          1.0 / (np.sqrt(25.0 * 3.0) * _s3),
          1.0 / (np.sqrt(25.0 * 5.0) * _s3))
    Wd = _mk_big(p['Wd0'][0], p['Wd1'][0], p['Wd2'][0], m_out, *sd)
    return Wq, Wk, Wv, Wd


def _group_mat():
    G = np.zeros((D, D), np.float32)
    for j in range(11):
        G[j, j] = 1.0
    for o in range(5):
        b = 11 + 3 * o
        G[b:b + 3, b:b + 3] = 1.0
    for o in range(5):
        b = 26 + 5 * o
        G[b:b + 5, b:b + 5] = 1.0
    for j in range(DIM_ACT, D):
        G[j, j] = 1.0
    return jnp.asarray(G)


_RB = 1024

_HP = lax.Precision.HIGHEST


def _dot(a, b):
    return lax.dot_general(a, b, (((1,), (0,)), ((), ())),
                           precision=_HP, preferred_element_type=jnp.float32)


def _norm_act(f, G):
    n2 = _dot(f * f, G)
    nrm = jnp.sqrt(n2 + 1e-10)
    return f * (jax.nn.sigmoid(nrm) / (nrm + 1e-5))


def _qkv_from_f(f, Wq, Wk, Wv, Wd, G):
    q = _norm_act(_dot(f, Wq), G)
    qt = _dot(q, Wd)
    k = _norm_act(_dot(f, Wk), G)
    v = _norm_act(_dot(f, Wv), G)
    return qt, k, v


def _embed_body(x_ref, ea_ref, We_ref, be_ref, Wq_ref, Wk_ref, Wv_ref,
                Wd_ref, G_ref, qt_ref, k_ref, v_ref):
    x = x_ref[...]
    ea = ea_ref[...]
    h = _dot(x, We_ref[...]) + be_ref[...]
    r = jnp.sqrt(jnp.sum(ea * ea, axis=1, keepdims=True) + 1e-12)
    u = ea / r
    ux, uy, uz = u[:, 0:1], u[:, 1:2], u[:, 2:3]
    sh = jnp.concatenate([
        jnp.ones_like(ux), _s3 * ux, _s3 * uy, _s3 * uz,
        _s15 * ux * uy, _s15 * uy * uz, (_s5 / 2.0) * (3.0 * uz * uz - 1.0),
        _s15 * ux * uz, (_s15 / 2.0) * (ux * ux - uy * uy)], axis=1)
    f = jnp.concatenate(
        [h, sh, jnp.zeros((x.shape[0], D - 19), jnp.float32)], axis=1)
    qt, k, v = _qkv_from_f(f, Wq_ref[...], Wk_ref[...], Wv_ref[...],
                           Wd_ref[...], G_ref[...])
    qt_ref[...] = qt
    k_ref[...] = k
    v_ref[...] = v


def _call_embed(xp, eap, We, be, Wq, Wk, Wv, Wd, G):
    row = pl.BlockSpec((_RB, D), lambda i: (i, 0))
    full = pl.BlockSpec(None, lambda i: (0, 0))
    out = jax.ShapeDtypeStruct((NP, D), jnp.float32)
    return pl.pallas_call(
        _embed_body,
        grid=(NP // _RB,),
        in_specs=[pl.BlockSpec((_RB, 10), lambda i: (i, 0)),
                  pl.BlockSpec((_RB, 3), lambda i: (i, 0)),
                  full, full, full, full, full, full, full],
        out_specs=[row, row, row],
        out_shape=[out, out, out],
    )(xp, eap, We, be, Wq, Wk, Wv, Wd, G)


def _dense_body(has_prev, *refs):
    if has_prev:
        (agg_ref, vp_ref, fp_ref, Wq_ref, Wk_ref, Wv_ref, Wd_ref, G_ref,
         f_ref, qt_ref, k_ref, v_ref) = refs
        f = agg_ref[...] + vp_ref[...] + fp_ref[...]
    else:
        (agg_ref, vp_ref, Wq_ref, Wk_ref, Wv_ref, Wd_ref, G_ref,
         f_ref, qt_ref, k_ref, v_ref) = refs
        f = agg_ref[...] + vp_ref[...]
    qt, k, v = _qkv_from_f(f, Wq_ref[...], Wk_ref[...], Wv_ref[...],
                           Wd_ref[...], G_ref[...])
    f_ref[...] = f
    qt_ref[...] = qt
    k_ref[...] = k
    v_ref[...] = v


def _call_dense(agg, vprev, fprev, Wq, Wk, Wv, Wd, G):
    has_prev = fprev is not None
    row = pl.BlockSpec((_RB, D), lambda i: (i, 0))
    full = pl.BlockSpec(None, lambda i: (0, 0))
    out = jax.ShapeDtypeStruct((NP, D), jnp.float32)
    args = [agg, vprev] + ([fprev] if has_prev else []) + [Wq, Wk, Wv, Wd, G]
    n_row = 3 if has_prev else 2
    return pl.pallas_call(
        functools.partial(_dense_body, has_prev),
        grid=(NP // _RB,),
        in_specs=[row] * n_row + [full] * 5,
        out_specs=[row, row, row, row],
        out_shape=[out, out, out, out],
    )(*args)


def _head_body(agg_ref, vp_ref, fp_ref, degp_ref, batch_ref, Wol_ref,
               W1_ref, b1_ref, W2_ref, b2_ref, Wo_ref, bo_ref,
               out_ref, acc_s, acc_c):
    i = pl.program_id(0)

    @pl.when(i == 0)
    def _():
        acc_s[...] = jnp.zeros_like(acc_s)
        acc_c[...] = jnp.zeros_like(acc_c)

    f = agg_ref[...] + vp_ref[...] + fp_ref[...]
    s = _dot(f, Wol_ref[...])
    nrm = jnp.sqrt(jnp.sum(s * s, axis=1, keepdims=True))
    s = s / jnp.maximum(nrm, 1e-12)
    s = jax.nn.relu(_dot(s, W1_ref[...]) + b1_ref[...])
    s = jax.nn.relu(_dot(s, W2_ref[...]) + b2_ref[...])
    deg = (degp_ref[0, :] + degp_ref[1, :]).reshape(-1, 1)
    gids = lax.broadcasted_iota(jnp.int32, (1, NG), 1)
    onehot = (batch_ref[...] == gids).astype(jnp.float32)
    ws = s * deg
    acc_s[...] += lax.dot_general(onehot, ws, (((0,), (0,)), ((), ())),
                                  precision=_HP,
                                  preferred_element_type=jnp.float32)
    acc_c[...] += lax.dot_general(
        onehot, jnp.broadcast_to(deg, ws.shape), (((0,), (0,)), ((), ())),
        precision=_HP, preferred_element_type=jnp.float32)

    @pl.when(i == pl.num_programs(0) - 1)
    def _():
        m = acc_s[...] / jnp.maximum(acc_c[...], 1.0)
        logits = _dot(m, Wo_ref[...]) + bo_ref[...]
        mx = jnp.max(logits, axis=1, keepdims=True)
        e = jnp.exp(logits - mx)
        out_ref[...] = e / jnp.sum(e, axis=1, keepdims=True)


def _call_head(agg, vprev, fprev, degp, batch2d, Wol, W1, b1, W2, b2, Wo, bo):
    row = pl.BlockSpec((_RB, D), lambda i: (i, 0))
    full = pl.BlockSpec(None, lambda i: (0, 0))
    return pl.pallas_call(
        _head_body,
        grid=(NP // _RB,),
        in_specs=[row, row, row,
                  pl.BlockSpec((2, _RB), lambda i: (0, i)),
                  pl.BlockSpec((_RB, 1), lambda i: (i, 0)),
                  full, full, full, full, full, full, full],
        out_specs=pl.BlockSpec(None, lambda i: (0, 0)),
        out_shape=jax.ShapeDtypeStruct((NG, 9), jnp.float32),
        scratch_shapes=[pltpu.VMEM((NG, HID), jnp.float32),
                        pltpu.VMEM((NG, HID), jnp.float32)],
    )(agg, vprev, fprev, degp, batch2d, Wol, W1, b1, W2, b2, Wo, bo)


_MESH = dict(core_axis_name="c", subcore_axis_name="s",
             num_cores=NCORES, num_subcores=NSUB)


def _lane():
    return lax.broadcasted_iota(jnp.int32, (16,), 0)


def _scores_body(with_deg, *refs):
    if with_deg:
        (qt_hbm, k_hbm, dst_hbm, src_hbm, ex_hbm, zp_hbm, degp_hbm,
         dst_v, src_v, qrows, krows, ex_v, stage_v, ones_v,
         z_sp, deg_sp, sem1, sem2) = refs
    else:
        (qt_hbm, k_hbm, dst_hbm, src_hbm, ex_hbm, zp_hbm,
         dst_v, src_v, qrows, krows, ex_v, stage_v,
         z_sp, sem1, sem2) = refs
    c = lax.axis_index("c")
    s = lax.axis_index("s")
    wid = s * NCORES + c
    zseg = NP // NSUB
    sz = pl.multiple_of(s * zseg, zseg)
    lane = _lane()

    def zero_loop(i, carry):
        stage_v[pl.ds(pl.multiple_of(i * 16, 16), 16)] = jnp.zeros(
            (16,), jnp.float32)
        return carry
    lax.fori_loop(0, zseg // 16, zero_loop, 0)
    pltpu.sync_copy(stage_v, z_sp.at[pl.ds(sz, zseg)])
    if with_deg:
        pltpu.sync_copy(stage_v, deg_sp.at[pl.ds(sz, zseg)])
    plsc.subcore_barrier()

    nchunk = EP // 32 // CHUNK
    base0 = wid * (EP // 32)

    def chunk_loop(ci, carry):
        base = pl.multiple_of(base0 + ci * CHUNK, CHUNK)
        ci1 = pltpu.async_copy(dst_hbm.at[pl.ds(base, CHUNK)], dst_v, sem1)
        ci2 = pltpu.async_copy(src_hbm.at[pl.ds(base, CHUNK)], src_v, sem2)
        ci1.wait()
        ci2.wait()
        cp1 = pltpu.async_copy(qt_hbm.at[dst_v], qrows, sem1)
        cp2 = pltpu.async_copy(k_hbm.at[src_v], krows, sem2)
        cp1.wait()
        cp2.wait()

        def grp_loop(g, carry2):
            o = pl.multiple_of(g * 16, 16)
            rows = o + lane
            acc = jnp.zeros((16,), jnp.float32)
            for j in range(DIM_ACT):
                colj = jnp.full((16,), j, jnp.int32)
                acc = acc + (plsc.load_gather(qrows, [rows, colj]) *
                             plsc.load_gather(krows, [rows, colj]))
            ex_v[pl.ds(o, 16)] = jnp.exp(acc)
            if with_deg:
                d16 = dst_v[pl.ds(o, 16)]
                ones_v[pl.ds(o, 16)] = jnp.where(
                    d16 < N, jnp.float32(1.0), jnp.float32(0.0))
            return carry2
        lax.fori_loop(0, GRP, grp_loop, 0)

        pltpu.sync_copy(ex_v, ex_hbm.at[pl.ds(base, CHUNK)])
        pltpu.sync_copy(ex_v, z_sp.at[dst_v], add=True)
        if with_deg:
            pltpu.sync_copy(ones_v, deg_sp.at[dst_v], add=True)
        return carry
    lax.fori_loop(0, nchunk, chunk_loop, 0)
    plsc.subcore_barrier()

    pltpu.sync_copy(z_sp.at[pl.ds(sz, zseg)],
                    zp_hbm.at[c, pl.ds(sz, zseg)])
    if with_deg:
        pltpu.sync_copy(deg_sp.at[pl.ds(sz, zseg)],
                        degp_hbm.at[c, pl.ds(sz, zseg)])


def _make_scores(with_deg):
    out_type = [jax.ShapeDtypeStruct((EP,), jnp.float32),
                jax.ShapeDtypeStruct((NCORES, NP), jnp.float32)]
    scratch = [pltpu.VMEM((CHUNK,), jnp.int32),
               pltpu.VMEM((CHUNK,), jnp.int32),
               pltpu.VMEM((CHUNK, D), jnp.float32),
               pltpu.VMEM((CHUNK, D), jnp.float32),
               pltpu.VMEM((CHUNK,), jnp.float32),
               pltpu.VMEM((NP // NSUB,), jnp.float32)]
    if with_deg:
        out_type.append(jax.ShapeDtypeStruct((NCORES, NP), jnp.float32))
        scratch.append(pltpu.VMEM((CHUNK,), jnp.float32))
    scratch.append(pltpu.VMEM_SHARED((NP,), jnp.float32))
    if with_deg:
        scratch.append(pltpu.VMEM_SHARED((NP,), jnp.float32))
    scratch += [pltpu.SemaphoreType.DMA, pltpu.SemaphoreType.DMA]
    return pl.kernel(
        functools.partial(_scores_body, with_deg),
        out_type=tuple(out_type),
        mesh=plsc.VectorSubcoreMesh(**_MESH),
        scratch_types=scratch,
        compiler_params=pltpu.CompilerParams(needs_layout_passes=False, use_tc_tiling_on_sc=False),
    )


@functools.cache
def _scores_deg():
    return _make_scores(True)


@functools.cache
def _scores():
    return _make_scores(False)


def _rsqrt_pos(x):
    i = plsc.bitcast(x, jnp.int32)
    i = jnp.int32(0x5F3759DF) - lax.shift_right_logical(i, 1)
    y = plsc.bitcast(i, jnp.float32)
    for _ in range(3):
        y = y * (1.5 - 0.5 * x * y * y)
    return y


def _agg_body(v_hbm, ex_hbm, zp_hbm, dst_hbm, src_hbm, agg_hbm,
              dst_v, src_v, ex_v, st_src, st_loc, st_w, vrows, zloc, ztmp,
              fout_sp, sem1):
    c = lax.axis_index("c")
    s = lax.axis_index("s")
    lane = _lane()
    lane_f = lane.astype(jnp.float32) * 0.0
    rows_per_tile = QUART // NSUB
    rbase = pl.multiple_of(s * rows_per_tile, rows_per_tile)
    nchunk = EP // NSUB // CHUNK_B
    base0 = s * (EP // NSUB)
    zero16f = jnp.zeros((16,), jnp.float32)
    zero16i = jnp.zeros((16,), jnp.int32)

    def fire():
        pltpu.async_copy(v_hbm.at[st_src], vrows, sem1).wait()

        def scale_loop(g, carry):
            o = pl.multiple_of(g * 16, 16)
            rows = o + lane
            w16 = st_w[pl.ds(o, 16)]
            for j in range(DIM_ACT):
                colj = jnp.full((16,), j, jnp.int32)
                vc = plsc.load_gather(vrows, [rows, colj])
                plsc.store_scatter(vrows, [rows, colj], vc * w16)
            return carry
        lax.fori_loop(0, FIRE // 16, scale_loop, 0)
        pltpu.sync_copy(vrows, fout_sp.at[st_loc], add=True)

    def pad_tail(cur):
        def pad_loop(g, carry):
            idx = g * 16 + lane
            m = (idx >= cur) & (idx < FIRE)
            plsc.store_scatter(st_src, [idx], zero16i, mask=m)
            plsc.store_scatter(st_loc, [idx], zero16i, mask=m)
            plsc.store_scatter(st_w, [idx], zero16f, mask=m)
            return carry
        lax.fori_loop(0, FIRE // 16, pad_loop, 0)

    for pi in range(NPASS):
        lo = pl.multiple_of((pi * NCORES + c) * QUART, QUART)

        pltpu.sync_copy(zp_hbm.at[0, pl.ds(lo, QUART)], zloc)
        pltpu.sync_copy(zp_hbm.at[1, pl.ds(lo, QUART)], ztmp)

        def zadd(i, carry):
            o = pl.multiple_of(i * 16, 16)
            zloc[pl.ds(o, 16)] = zloc[pl.ds(o, 16)] + ztmp[pl.ds(o, 16)]
            return carry
        lax.fori_loop(0, QUART // 16, zadd, 0)

        def vz(i, carry):
            for cj in range(D // 16):
                vrows[i, pl.ds(cj * 16, 16)] = zero16f
            return carry
        lax.fori_loop(0, FIRE, vz, 0)
        done = 0
        while done < rows_per_tile:
            step = min(FIRE, rows_per_tile - done)
            pltpu.sync_copy(vrows.at[pl.ds(0, step)],
                            fout_sp.at[pl.ds(rbase + done, step)])
            done += step
        plsc.subcore_barrier()

        def chunk_loop(ci, cur):
            base = pl.multiple_of(base0 + ci * CHUNK_B, CHUNK_B)
            cp1 = pltpu.async_copy(dst_hbm.at[pl.ds(base, CHUNK_B)], dst_v,
                                   sem1)
            cp2 = pltpu.async_copy(src_hbm.at[pl.ds(base, CHUNK_B)], src_v,
                                   sem1)
            cp3 = pltpu.async_copy(ex_hbm.at[pl.ds(base, CHUNK_B)], ex_v,
                                   sem1)
            cp1.wait()
            cp2.wait()
            cp3.wait()

            def grp_loop(g, cur2):
                o = pl.multiple_of(g * 16, 16)
                d16 = dst_v[pl.ds(o, 16)]
                loc = d16 - lo
                owned = (loc >= 0) & (loc < QUART)
                locc = jnp.clip(loc, 0, QUART - 1)
                zv = plsc.load_gather(zloc, [locc])
                exv = ex_v[pl.ds(o, 16)]
                ratio = jnp.where(
                    zv > 0.0, exv / jnp.where(zv > 0.0, zv, 1.0), 0.0)
                w = jnp.where(ratio > 0.0, ratio * _rsqrt_pos(ratio), 0.0)
                ranks = plsc.cumsum(owned.astype(jnp.int32))
                n = jnp.sum(owned.astype(jnp.int32), axis=0)
                pos = jnp.clip(cur2 + ranks - 1, 0, FIRE - 1)
                plsc.store_scatter(st_src, [pos], src_v[pl.ds(o, 16)],
                                   mask=owned)
                plsc.store_scatter(st_loc, [pos], locc, mask=owned)
                plsc.store_scatter(st_w, [pos], w, mask=owned)
                cur3 = cur2 + n

                @pl.when(cur3 > FIRE - 16)
                def _():
                    pad_tail(cur3)
                    fire()
                return jnp.where(cur3 > FIRE - 16, 0, cur3)
            return lax.fori_loop(0, GRP_B, grp_loop, cur)
        cur = lax.fori_loop(0, nchunk, chunk_loop, jnp.int32(0))

        pad_tail(cur)
        fire()
        plsc.subcore_barrier()

        pltpu.sync_copy(fout_sp.at[pl.ds(rbase, rows_per_tile)],
                        agg_hbm.at[pl.ds(lo + rbase, rows_per_tile)])


@functools.cache
def _agg():
    return pl.kernel(
        _agg_body,
        out_type=jax.ShapeDtypeStruct((NP, D), jnp.float32),
        mesh=plsc.VectorSubcoreMesh(**_MESH),
        scratch_types=[pltpu.VMEM((CHUNK_B,), jnp.int32),
                       pltpu.VMEM((CHUNK_B,), jnp.int32),
                       pltpu.VMEM((CHUNK_B,), jnp.float32),
                       pltpu.VMEM((FIRE,), jnp.int32),
                       pltpu.VMEM((FIRE,), jnp.int32),
                       pltpu.VMEM((FIRE,), jnp.float32),
                       pltpu.VMEM((FIRE, D), jnp.float32),
                       pltpu.VMEM((QUART,), jnp.float32),
                       pltpu.VMEM((QUART,), jnp.float32),
                       pltpu.VMEM_SHARED((QUART, D), jnp.float32),
                       pltpu.SemaphoreType.DMA],
        compiler_params=pltpu.CompilerParams(
            needs_layout_passes=False, use_tc_tiling_on_sc=False),
    )


def kernel(x, edge_attr, params, edge_index, batch):
    p = params
    src = edge_index[0].astype(jnp.int32)
    dst = edge_index[1].astype(jnp.int32)

    xp = jnp.pad(x, ((0, NP - N), (0, 0)))
    eap = jnp.pad(edge_attr, ((0, NP - N), (0, 0)))
    padE = jnp.full((EP - E,), PAD_NODE, jnp.int32)
    dst_p = jnp.concatenate([dst, padE])
    src_p = jnp.concatenate([src, padE])
    batch2d = jnp.pad(batch.astype(jnp.int32), (0, NP - N)).reshape(NP, 1)

    G = _group_mat()
    Wq0, Wk0, Wv0, Wd0 = _layer_bigs(p['et'], (11, 1, 1))
    Wq1, Wk1, Wv1, Wd1 = _layer_bigs(p['m_et'][0], (11, 5, 5))
    Wq2, Wk2, Wv2, Wd2 = _layer_bigs(p['m_et'][1], (11, 5, 5))
    be = p['b_embd'].reshape(1, 10)
    Wol = jnp.zeros((D, HID), jnp.float32).at[:11, :].set(
        p['W_ol'] / np.sqrt(11.0))
    (W1h, b1h), (W2h, b2h) = p['lin']
    b1h = b1h.reshape(1, HID)
    b2h = b2h.reshape(1, HID)
    bo = p['b_out'].reshape(1, 9)

    qt, k, v = _call_embed(xp, eap, p['W_embd'], be, Wq0, Wk0, Wv0, Wd0, G)
    ex, zp, degp = _scores_deg()(qt, k, dst_p, src_p)
    agg = _agg()(v, ex, zp, dst_p, src_p)

    f1, qt, k, v1 = _call_dense(agg, v, None, Wq1, Wk1, Wv1, Wd1, G)
    ex, zp = _scores()(qt, k, dst_p, src_p)
    agg = _agg()(v1, ex, zp, dst_p, src_p)

    f2, qt, k, v2 = _call_dense(agg, v1, f1, Wq2, Wk2, Wv2, Wd2, G)
    ex, zp = _scores()(qt, k, dst_p, src_p)
    agg = _agg()(v2, ex, zp, dst_p, src_p)

    return _call_head(agg, v2, f2, degp, batch2d,
                      Wol, W1h, b1h, W2h, b2h, p['W_out'], bo)


# speedup vs baseline: 4.2764x; 1.0398x over previous
import functools

import numpy as np
import jax
import jax.numpy as jnp
from jax import lax
from jax.experimental import pallas as pl
from jax.experimental.pallas import tpu as pltpu, tpu_sc as plsc

N = 50000
NP = 51200
E = 800000
EP = 802816
D = 64
NG = 64
HID = 128
HALF = NP // 2
CHUNK = 512
GRP = CHUNK // 16
PAD_NODE = 50008
NCORES = 2
NSUB = 16
DIM_ACT = 51
NPASS = 4
QUART = NP // (NCORES * NPASS)
FIRE = 512
CHUNK_A = 256
CHUNK_B = 3584
GRP_B = CHUNK_B // 16

_s3 = float(np.sqrt(3.0))
_s5 = float(np.sqrt(5.0))
_s15 = float(np.sqrt(15.0))


def _mk_big(W0, W1, W2, m_in, s0, s1, s2):
    m0, m1, m2 = m_in
    o0, o1, o2 = W0.shape[1], W1.shape[1], W2.shape[1]
    Wb = jnp.zeros((D, D), jnp.float32)
    Wb = Wb.at[:m0, :o0].set(W0 * s0)
    r, c = m0, o0
    Wb = Wb.at[r:r + 3 * m1, c:c + 3 * o1].set(jnp.kron(W1, jnp.eye(3)) * s1)
    r, c = r + 3 * m1, c + 3 * o1
    Wb = Wb.at[r:r + 5 * m2, c:c + 5 * o2].set(jnp.kron(W2, jnp.eye(5)) * s2)
    return Wb


def _layer_bigs(p, m_in):
    i0, i1, i2 = m_in
    sq = (1.0 / np.sqrt(i0), 1.0 / np.sqrt(i1), 1.0 / np.sqrt(i2))
    Wq = _mk_big(p['Wq0'], p['Wq1'], p['Wq2'], m_in, *sq)
    Wk = _mk_big(p['Wk0'], p['Wk1'], p['Wk2'], m_in, *sq)
    Wv = _mk_big(p['Wv0'], p['Wv1'], p['Wv2'], m_in, *sq)
    m_out = (11, 5, 5)
    sd = (1.0 / (11.0 * _s3),
          1.0 / (np.sqrt(25.0 * 3.0) * _s3),
          1.0 / (np.sqrt(25.0 * 5.0) * _s3))
    Wd = _mk_big(p['Wd0'][0], p['Wd1'][0], p['Wd2'][0], m_out, *sd)
    return Wq, Wk, Wv, Wd


def _group_mat():
    G = np.zeros((D, D), np.float32)
    for j in range(11):
        G[j, j] = 1.0
    for o in range(5):
        b = 11 + 3 * o
        G[b:b + 3, b:b + 3] = 1.0
    for o in range(5):
        b = 26 + 5 * o
        G[b:b + 5, b:b + 5] = 1.0
    for j in range(DIM_ACT, D):
        G[j, j] = 1.0
    return jnp.asarray(G)


_RB = 1024

_HP = lax.Precision.HIGHEST


def _dot(a, b):
    return lax.dot_general(a, b, (((1,), (0,)), ((), ())),
                           precision=_HP, preferred_element_type=jnp.float32)


def _norm_act(f, G):
    n2 = _dot(f * f, G)
    nrm = jnp.sqrt(n2 + 1e-10)
    return f * (jax.nn.sigmoid(nrm) / (nrm + 1e-5))


def _qkv_from_f(f, Wq, Wk, Wv, Wd, G):
    q = _norm_act(_dot(f, Wq), G)
    qt = _dot(q, Wd)
    k = _norm_act(_dot(f, Wk), G)
    v = _norm_act(_dot(f, Wv), G)
    return qt, k, v


def _embed_body(x_ref, ea_ref, We_ref, be_ref, Wq_ref, Wk_ref, Wv_ref,
                Wd_ref, G_ref, qt_ref, k_ref, v_ref):
    x = x_ref[...]
    ea = ea_ref[...]
    h = _dot(x, We_ref[...]) + be_ref[...]
    r = jnp.sqrt(jnp.sum(ea * ea, axis=1, keepdims=True) + 1e-12)
    u = ea / r
    ux, uy, uz = u[:, 0:1], u[:, 1:2], u[:, 2:3]
    sh = jnp.concatenate([
        jnp.ones_like(ux), _s3 * ux, _s3 * uy, _s3 * uz,
        _s15 * ux * uy, _s15 * uy * uz, (_s5 / 2.0) * (3.0 * uz * uz - 1.0),
        _s15 * ux * uz, (_s15 / 2.0) * (ux * ux - uy * uy)], axis=1)
    f = jnp.concatenate(
        [h, sh, jnp.zeros((x.shape[0], D - 19), jnp.float32)], axis=1)
    qt, k, v = _qkv_from_f(f, Wq_ref[...], Wk_ref[...], Wv_ref[...],
                           Wd_ref[...], G_ref[...])
    qt_ref[...] = qt
    k_ref[...] = k
    v_ref[...] = v


def _call_embed(xp, eap, We, be, Wq, Wk, Wv, Wd, G):
    row = pl.BlockSpec((_RB, D), lambda i: (i, 0))
    full = pl.BlockSpec(None, lambda i: (0, 0))
    out = jax.ShapeDtypeStruct((NP, D), jnp.float32)
    return pl.pallas_call(
        _embed_body,
        grid=(NP // _RB,),
        in_specs=[pl.BlockSpec((_RB, 10), lambda i: (i, 0)),
                  pl.BlockSpec((_RB, 3), lambda i: (i, 0)),
                  full, full, full, full, full, full, full],
        out_specs=[row, row, row],
        out_shape=[out, out, out],
    )(xp, eap, We, be, Wq, Wk, Wv, Wd, G)


def _dense_body(has_prev, *refs):
    if has_prev:
        (agg_ref, vp_ref, fp_ref, Wq_ref, Wk_ref, Wv_ref, Wd_ref, G_ref,
         f_ref, qt_ref, k_ref, v_ref) = refs
        f = agg_ref[...] + vp_ref[...] + fp_ref[...]
    else:
        (agg_ref, vp_ref, Wq_ref, Wk_ref, Wv_ref, Wd_ref, G_ref,
         f_ref, qt_ref, k_ref, v_ref) = refs
        f = agg_ref[...] + vp_ref[...]
    qt, k, v = _qkv_from_f(f, Wq_ref[...], Wk_ref[...], Wv_ref[...],
                           Wd_ref[...], G_ref[...])
    f_ref[...] = f
    qt_ref[...] = qt
    k_ref[...] = k
    v_ref[...] = v


def _call_dense(agg, vprev, fprev, Wq, Wk, Wv, Wd, G):
    has_prev = fprev is not None
    row = pl.BlockSpec((_RB, D), lambda i: (i, 0))
    full = pl.BlockSpec(None, lambda i: (0, 0))
    out = jax.ShapeDtypeStruct((NP, D), jnp.float32)
    args = [agg, vprev] + ([fprev] if has_prev else []) + [Wq, Wk, Wv, Wd, G]
    n_row = 3 if has_prev else 2
    return pl.pallas_call(
        functools.partial(_dense_body, has_prev),
        grid=(NP // _RB,),
        in_specs=[row] * n_row + [full] * 5,
        out_specs=[row, row, row, row],
        out_shape=[out, out, out, out],
    )(*args)


def _head_body(agg_ref, vp_ref, fp_ref, degp_ref, batch_ref, Wol_ref,
               W1_ref, b1_ref, W2_ref, b2_ref, Wo_ref, bo_ref,
               out_ref, acc_s, acc_c):
    i = pl.program_id(0)

    @pl.when(i == 0)
    def _():
        acc_s[...] = jnp.zeros_like(acc_s)
        acc_c[...] = jnp.zeros_like(acc_c)

    f = agg_ref[...] + vp_ref[...] + fp_ref[...]
    s = _dot(f, Wol_ref[...])
    nrm = jnp.sqrt(jnp.sum(s * s, axis=1, keepdims=True))
    s = s / jnp.maximum(nrm, 1e-12)
    s = jax.nn.relu(_dot(s, W1_ref[...]) + b1_ref[...])
    s = jax.nn.relu(_dot(s, W2_ref[...]) + b2_ref[...])
    deg = (degp_ref[0, :] + degp_ref[1, :]).reshape(-1, 1)
    gids = lax.broadcasted_iota(jnp.int32, (1, NG), 1)
    onehot = (batch_ref[...] == gids).astype(jnp.float32)
    ws = s * deg
    acc_s[...] += lax.dot_general(onehot, ws, (((0,), (0,)), ((), ())),
                                  precision=_HP,
                                  preferred_element_type=jnp.float32)
    acc_c[...] += lax.dot_general(
        onehot, jnp.broadcast_to(deg, ws.shape), (((0,), (0,)), ((), ())),
        precision=_HP, preferred_element_type=jnp.float32)

    @pl.when(i == pl.num_programs(0) - 1)
    def _():
        m = acc_s[...] / jnp.maximum(acc_c[...], 1.0)
        logits = _dot(m, Wo_ref[...]) + bo_ref[...]
        mx = jnp.max(logits, axis=1, keepdims=True)
        e = jnp.exp(logits - mx)
        out_ref[...] = e / jnp.sum(e, axis=1, keepdims=True)


def _call_head(agg, vprev, fprev, degp, batch2d, Wol, W1, b1, W2, b2, Wo, bo):
    row = pl.BlockSpec((_RB, D), lambda i: (i, 0))
    full = pl.BlockSpec(None, lambda i: (0, 0))
    return pl.pallas_call(
        _head_body,
        grid=(NP // _RB,),
        in_specs=[row, row, row,
                  pl.BlockSpec((2, _RB), lambda i: (0, i)),
                  pl.BlockSpec((_RB, 1), lambda i: (i, 0)),
                  full, full, full, full, full, full, full],
        out_specs=pl.BlockSpec(None, lambda i: (0, 0)),
        out_shape=jax.ShapeDtypeStruct((NG, 9), jnp.float32),
        scratch_shapes=[pltpu.VMEM((NG, HID), jnp.float32),
                        pltpu.VMEM((NG, HID), jnp.float32)],
    )(agg, vprev, fprev, degp, batch2d, Wol, W1, b1, W2, b2, Wo, bo)


_MESH = dict(core_axis_name="c", subcore_axis_name="s",
             num_cores=NCORES, num_subcores=NSUB)


def _lane():
    return lax.broadcasted_iota(jnp.int32, (16,), 0)


def _scores_body(with_deg, *refs):
    if with_deg:
        (qt_hbm, k_hbm, dst_hbm, src_hbm, ex_hbm, zp_hbm, degp_hbm,
         dst_v, src_v, qrows, krows, ex_v, stage_v, ones_v,
         z_sp, deg_sp, semq0, semq1, semk0, semk1) = refs
    else:
        (qt_hbm, k_hbm, dst_hbm, src_hbm, ex_hbm, zp_hbm,
         dst_v, src_v, qrows, krows, ex_v, stage_v,
         z_sp, semq0, semq1, semk0, semk1) = refs
    semq = (semq0, semq1)
    semk = (semk0, semk1)
    c = lax.axis_index("c")
    s = lax.axis_index("s")
    wid = s * NCORES + c
    zseg = NP // NSUB
    sz = pl.multiple_of(s * zseg, zseg)
    lane = _lane()

    def zero_loop(i, carry):
        stage_v[pl.ds(pl.multiple_of(i * 16, 16), 16)] = jnp.zeros(
            (16,), jnp.float32)
        return carry
    lax.fori_loop(0, zseg // 16, zero_loop, 0)
    pltpu.sync_copy(stage_v, z_sp.at[pl.ds(sz, zseg)])
    if with_deg:
        pltpu.sync_copy(stage_v, deg_sp.at[pl.ds(sz, zseg)])
    plsc.subcore_barrier()

    nchunk = EP // 32 // CHUNK_A
    base0 = wid * (EP // 32)

    def start_chunk(ci, b):
        base = pl.multiple_of(base0 + ci * CHUNK_A, CHUNK_A)
        i1 = pltpu.async_copy(dst_hbm.at[pl.ds(base, CHUNK_A)],
                              dst_v.at[b], semq[b])
        i2 = pltpu.async_copy(src_hbm.at[pl.ds(base, CHUNK_A)],
                              src_v.at[b], semk[b])
        i1.wait()
        i2.wait()
        pltpu.async_copy(qt_hbm.at[dst_v.at[b]], qrows.at[b], semq[b])
        pltpu.async_copy(k_hbm.at[src_v.at[b]], krows.at[b], semk[b])

    def drain_chunk(b):
        pltpu.make_async_copy(qt_hbm.at[dst_v.at[b]], qrows.at[b],
                              semq[b]).wait()
        pltpu.make_async_copy(k_hbm.at[src_v.at[b]], krows.at[b],
                              semk[b]).wait()

    start_chunk(jnp.int32(0), 0)

    def pair_loop(pi, carry):
        for half in range(2):
            ci = pi * 2 + half
            b = half
            other = 1 - half
            drain_chunk(b)

            @pl.when(ci < nchunk - 1)
            def _():
                start_chunk(ci + 1, other)

            def grp_loop(g, carry2):
                o = pl.multiple_of(g * 16, 16)
                rows = o + lane
                acc = jnp.zeros((16,), jnp.float32)
                for j in range(DIM_ACT):
                    colj = jnp.full((16,), j, jnp.int32)
                    acc = acc + (plsc.load_gather(qrows.at[b],
                                                  [rows, colj]) *
                                 plsc.load_gather(krows.at[b],
                                                  [rows, colj]))
                ex_v[b, pl.ds(o, 16)] = jnp.exp(acc)
                if with_deg:
                    d16 = dst_v[b, pl.ds(o, 16)]
                    ones_v[b, pl.ds(o, 16)] = jnp.where(
                        d16 < N, jnp.float32(1.0), jnp.float32(0.0))
                return carry2
            lax.fori_loop(0, CHUNK_A // 16, grp_loop, 0)

            base = pl.multiple_of(base0 + ci * CHUNK_A, CHUNK_A)
            pltpu.sync_copy(ex_v.at[b], ex_hbm.at[pl.ds(base, CHUNK_A)])
            pltpu.sync_copy(ex_v.at[b], z_sp.at[dst_v.at[b]], add=True)
            if with_deg:
                pltpu.sync_copy(ones_v.at[b], deg_sp.at[dst_v.at[b]],
                                add=True)
        return carry
    lax.fori_loop(0, nchunk // 2, pair_loop, 0)
    plsc.subcore_barrier()

    pltpu.sync_copy(z_sp.at[pl.ds(sz, zseg)],
                    zp_hbm.at[c, pl.ds(sz, zseg)])
    if with_deg:
        pltpu.sync_copy(deg_sp.at[pl.ds(sz, zseg)],
                        degp_hbm.at[c, pl.ds(sz, zseg)])


def _make_scores(with_deg):
    out_type = [jax.ShapeDtypeStruct((EP,), jnp.float32),
                jax.ShapeDtypeStruct((NCORES, NP), jnp.float32)]
    scratch = [pltpu.VMEM((2, CHUNK_A), jnp.int32),
               pltpu.VMEM((2, CHUNK_A), jnp.int32),
               pltpu.VMEM((2, CHUNK_A, D), jnp.float32),
               pltpu.VMEM((2, CHUNK_A, D), jnp.float32),
               pltpu.VMEM((2, CHUNK_A), jnp.float32),
               pltpu.VMEM((NP // NSUB,), jnp.float32)]
    if with_deg:
        out_type.append(jax.ShapeDtypeStruct((NCORES, NP), jnp.float32))
        scratch.append(pltpu.VMEM((2, CHUNK_A), jnp.float32))
    scratch.append(pltpu.VMEM_SHARED((NP,), jnp.float32))
    if with_deg:
        scratch.append(pltpu.VMEM_SHARED((NP,), jnp.float32))
    scratch += [pltpu.SemaphoreType.DMA, pltpu.SemaphoreType.DMA,
                pltpu.SemaphoreType.DMA, pltpu.SemaphoreType.DMA]
    return pl.kernel(
        functools.partial(_scores_body, with_deg),
        out_type=tuple(out_type),
        mesh=plsc.VectorSubcoreMesh(**_MESH),
        scratch_types=scratch,
        compiler_params=pltpu.CompilerParams(needs_layout_passes=False, use_tc_tiling_on_sc=False),
    )


@functools.cache
def _scores_deg():
    return _make_scores(True)


@functools.cache
def _scores():
    return _make_scores(False)


def _rsqrt_pos(x):
    i = plsc.bitcast(x, jnp.int32)
    i = jnp.int32(0x5F3759DF) - lax.shift_right_logical(i, 1)
    y = plsc.bitcast(i, jnp.float32)
    for _ in range(3):
        y = y * (1.5 - 0.5 * x * y * y)
    return y


def _agg_body(v_hbm, ex_hbm, zp_hbm, dst_hbm, src_hbm, agg_hbm,
              dst_v, src_v, ex_v, st_src, st_loc, st_w, vrows, zloc, ztmp,
              fout_sp, sem1):
    c = lax.axis_index("c")
    s = lax.axis_index("s")
    lane = _lane()
    rows_per_tile = QUART // NSUB
    rbase = pl.multiple_of(s * rows_per_tile, rows_per_tile)
    nchunk = EP // NSUB // CHUNK_B
    base0 = s * (EP // NSUB)
    zero16f = jnp.zeros((16,), jnp.float32)
    zero16i = jnp.zeros((16,), jnp.int32)

    def fire():
        pltpu.async_copy(v_hbm.at[st_src], vrows, sem1).wait()

        def scale_loop(g, carry):
            o = pl.multiple_of(g * 16, 16)
            rows = o + lane
            w16 = st_w[pl.ds(o, 16)]
            for j in range(DIM_ACT):
                colj = jnp.full((16,), j, jnp.int32)
                vc = plsc.load_gather(vrows, [rows, colj])
                plsc.store_scatter(vrows, [rows, colj], vc * w16)
            return carry
        lax.fori_loop(0, FIRE // 16, scale_loop, 0)
        pltpu.sync_copy(vrows, fout_sp.at[st_loc], add=True)

    def pad_tail(cur):
        def pad_loop(g, carry):
            idx = g * 16 + lane
            m = (idx >= cur) & (idx < FIRE)
            plsc.store_scatter(st_src, [idx], zero16i, mask=m)
            plsc.store_scatter(st_loc, [idx], zero16i, mask=m)
            plsc.store_scatter(st_w, [idx], zero16f, mask=m)
            return carry
        lax.fori_loop(0, FIRE // 16, pad_loop, 0)

    for pi in range(NPASS):
        lo = pl.multiple_of((pi * NCORES + c) * QUART, QUART)

        pltpu.sync_copy(zp_hbm.at[0, pl.ds(lo, QUART)], zloc)
        pltpu.sync_copy(zp_hbm.at[1, pl.ds(lo, QUART)], ztmp)

        def zadd(i, carry):
            o = pl.multiple_of(i * 16, 16)
            zloc[pl.ds(o, 16)] = zloc[pl.ds(o, 16)] + ztmp[pl.ds(o, 16)]
            return carry
        lax.fori_loop(0, QUART // 16, zadd, 0)

        def vz(i, carry):
            for cj in range(D // 16):
                vrows[i, pl.ds(cj * 16, 16)] = zero16f
            return carry
        lax.fori_loop(0, FIRE, vz, 0)
        done = 0
        while done < rows_per_tile:
            step = min(FIRE, rows_per_tile - done)
            pltpu.sync_copy(vrows.at[pl.ds(0, step)],
                            fout_sp.at[pl.ds(rbase + done, step)])
            done += step
        plsc.subcore_barrier()

        def chunk_loop(ci, cur):
            base = pl.multiple_of(base0 + ci * CHUNK_B, CHUNK_B)
            cp1 = pltpu.async_copy(dst_hbm.at[pl.ds(base, CHUNK_B)], dst_v,
                                   sem1)
            cp2 = pltpu.async_copy(src_hbm.at[pl.ds(base, CHUNK_B)], src_v,
                                   sem1)
            cp3 = pltpu.async_copy(ex_hbm.at[pl.ds(base, CHUNK_B)], ex_v,
                                   sem1)
            cp1.wait()
            cp2.wait()
            cp3.wait()

            def grp_loop(g, cur2):
                o = pl.multiple_of(g * 16, 16)
                d16 = dst_v[pl.ds(o, 16)]
                loc = d16 - lo
                owned = (loc >= 0) & (loc < QUART)
                locc = jnp.clip(loc, 0, QUART - 1)
                zv = plsc.load_gather(zloc, [locc])
                exv = ex_v[pl.ds(o, 16)]
                ratio = jnp.where(
                    zv > 0.0, exv / jnp.where(zv > 0.0, zv, 1.0), 0.0)
                w = jnp.where(ratio > 0.0, ratio * _rsqrt_pos(ratio), 0.0)
                ranks = plsc.cumsum(owned.astype(jnp.int32))
                n = jnp.sum(owned.astype(jnp.int32), axis=0)
                pos = jnp.clip(cur2 + ranks - 1, 0, FIRE - 1)
                plsc.store_scatter(st_src, [pos], src_v[pl.ds(o, 16)],
                                   mask=owned)
                plsc.store_scatter(st_loc, [pos], locc, mask=owned)
                plsc.store_scatter(st_w, [pos], w, mask=owned)
                cur3 = cur2 + n

                @pl.when(cur3 > FIRE - 16)
                def _():
                    pad_tail(cur3)
                    fire()
                return jnp.where(cur3 > FIRE - 16, 0, cur3)
            return lax.fori_loop(0, GRP_B, grp_loop, cur)
        cur = lax.fori_loop(0, nchunk, chunk_loop, jnp.int32(0))

        pad_tail(cur)
        fire()
        plsc.subcore_barrier()

        pltpu.sync_copy(fout_sp.at[pl.ds(rbase, rows_per_tile)],
                        agg_hbm.at[pl.ds(lo + rbase, rows_per_tile)])


@functools.cache
def _agg():
    return pl.kernel(
        _agg_body,
        out_type=jax.ShapeDtypeStruct((NP, D), jnp.float32),
        mesh=plsc.VectorSubcoreMesh(**_MESH),
        scratch_types=[pltpu.VMEM((CHUNK_B,), jnp.int32),
                       pltpu.VMEM((CHUNK_B,), jnp.int32),
                       pltpu.VMEM((CHUNK_B,), jnp.float32),
                       pltpu.VMEM((FIRE,), jnp.int32),
                       pltpu.VMEM((FIRE,), jnp.int32),
                       pltpu.VMEM((FIRE,), jnp.float32),
                       pltpu.VMEM((FIRE, D), jnp.float32),
                       pltpu.VMEM((QUART,), jnp.float32),
                       pltpu.VMEM((QUART,), jnp.float32),
                       pltpu.VMEM_SHARED((QUART, D), jnp.float32),
                       pltpu.SemaphoreType.DMA],
        compiler_params=pltpu.CompilerParams(
            needs_layout_passes=False, use_tc_tiling_on_sc=False),
    )


def kernel(x, edge_attr, params, edge_index, batch):
    p = params
    src = edge_index[0].astype(jnp.int32)
    dst = edge_index[1].astype(jnp.int32)

    xp = jnp.pad(x, ((0, NP - N), (0, 0)))
    eap = jnp.pad(edge_attr, ((0, NP - N), (0, 0)))
    padE = jnp.full((EP - E,), PAD_NODE, jnp.int32)
    dst_p = jnp.concatenate([dst, padE])
    src_p = jnp.concatenate([src, padE])
    batch2d = jnp.pad(batch.astype(jnp.int32), (0, NP - N)).reshape(NP, 1)

    G = _group_mat()
    Wq0, Wk0, Wv0, Wd0 = _layer_bigs(p['et'], (11, 1, 1))
    Wq1, Wk1, Wv1, Wd1 = _layer_bigs(p['m_et'][0], (11, 5, 5))
    Wq2, Wk2, Wv2, Wd2 = _layer_bigs(p['m_et'][1], (11, 5, 5))
    be = p['b_embd'].reshape(1, 10)
    Wol = jnp.zeros((D, HID), jnp.float32).at[:11, :].set(
        p['W_ol'] / np.sqrt(11.0))
    (W1h, b1h), (W2h, b2h) = p['lin']
    b1h = b1h.reshape(1, HID)
    b2h = b2h.reshape(1, HID)
    bo = p['b_out'].reshape(1, 9)

    qt, k, v = _call_embed(xp, eap, p['W_embd'], be, Wq0, Wk0, Wv0, Wd0, G)
    ex, zp, degp = _scores_deg()(qt, k, dst_p, src_p)
    agg = _agg()(v, ex, zp, dst_p, src_p)

    f1, qt, k, v1 = _call_dense(agg, v, None, Wq1, Wk1, Wv1, Wd1, G)
    ex, zp = _scores()(qt, k, dst_p, src_p)
    agg = _agg()(v1, ex, zp, dst_p, src_p)

    f2, qt, k, v2 = _call_dense(agg, v1, f1, Wq2, Wk2, Wv2, Wd2, G)
    ex, zp = _scores()(qt, k, dst_p, src_p)
    agg = _agg()(v2, ex, zp, dst_p, src_p)

    return _call_head(agg, v2, f2, degp, batch2d,
                      Wol, W1h, b1h, W2h, b2h, p['W_out'], bo)


# speedup vs baseline: 4.3206x; 1.0103x over previous
import functools

import numpy as np
import jax
import jax.numpy as jnp
from jax import lax
from jax.experimental import pallas as pl
from jax.experimental.pallas import tpu as pltpu, tpu_sc as plsc

N = 50000
NP = 51200
E = 800000
EP = 802816
D = 64
NG = 64
HID = 128
HALF = NP // 2
CHUNK = 512
GRP = CHUNK // 16
PAD_NODE = 50008
NCORES = 2
NSUB = 16
DIM_ACT = 51
NPASS = 4
QUART = NP // (NCORES * NPASS)
FIRE = 512
CHUNK_A = 256
CHUNK_B = 3584
GRP_B = CHUNK_B // 16

_s3 = float(np.sqrt(3.0))
_s5 = float(np.sqrt(5.0))
_s15 = float(np.sqrt(15.0))


def _mk_big(W0, W1, W2, m_in, s0, s1, s2):
    m0, m1, m2 = m_in
    o0, o1, o2 = W0.shape[1], W1.shape[1], W2.shape[1]
    Wb = jnp.zeros((D, D), jnp.float32)
    Wb = Wb.at[:m0, :o0].set(W0 * s0)
    r, c = m0, o0
    Wb = Wb.at[r:r + 3 * m1, c:c + 3 * o1].set(jnp.kron(W1, jnp.eye(3)) * s1)
    r, c = r + 3 * m1, c + 3 * o1
    Wb = Wb.at[r:r + 5 * m2, c:c + 5 * o2].set(jnp.kron(W2, jnp.eye(5)) * s2)
    return Wb


def _layer_bigs(p, m_in):
    i0, i1, i2 = m_in
    sq = (1.0 / np.sqrt(i0), 1.0 / np.sqrt(i1), 1.0 / np.sqrt(i2))
    Wq = _mk_big(p['Wq0'], p['Wq1'], p['Wq2'], m_in, *sq)
    Wk = _mk_big(p['Wk0'], p['Wk1'], p['Wk2'], m_in, *sq)
    Wv = _mk_big(p['Wv0'], p['Wv1'], p['Wv2'], m_in, *sq)
    m_out = (11, 5, 5)
    sd = (1.0 / (11.0 * _s3),
          1.0 / (np.sqrt(25.0 * 3.0) * _s3),
          1.0 / (np.sqrt(25.0 * 5.0) * _s3))
    Wd = _mk_big(p['Wd0'][0], p['Wd1'][0], p['Wd2'][0], m_out, *sd)
    return Wq, Wk, Wv, Wd


def _group_mat():
    G = np.zeros((D, D), np.float32)
    for j in range(11):
        G[j, j] = 1.0
    for o in range(5):
        b = 11 + 3 * o
        G[b:b + 3, b:b + 3] = 1.0
    for o in range(5):
        b = 26 + 5 * o
        G[b:b + 5, b:b + 5] = 1.0
    for j in range(DIM_ACT, D):
        G[j, j] = 1.0
    return jnp.asarray(G)


_RB = 1024

_HP = lax.Precision.HIGHEST


def _dot(a, b):
    return lax.dot_general(a, b, (((1,), (0,)), ((), ())),
                           precision=_HP, preferred_element_type=jnp.float32)


def _norm_act(f, G):
    n2 = _dot(f * f, G)
    nrm = jnp.sqrt(n2 + 1e-10)
    return f * (jax.nn.sigmoid(nrm) / (nrm + 1e-5))


def _qkv_from_f(f, Wq, Wk, Wv, Wd, G):
    q = _norm_act(_dot(f, Wq), G)
    qt = _dot(q, Wd)
    k = _norm_act(_dot(f, Wk), G)
    v = _norm_act(_dot(f, Wv), G)
    return qt, k, v


def _embed_body(x_ref, ea_ref, We_ref, be_ref, Wq_ref, Wk_ref, Wv_ref,
                Wd_ref, G_ref, qt_ref, k_ref, v_ref):
    x = x_ref[...]
    ea = ea_ref[...]
    h = _dot(x, We_ref[...]) + be_ref[...]
    r = jnp.sqrt(jnp.sum(ea * ea, axis=1, keepdims=True) + 1e-12)
    u = ea / r
    ux, uy, uz = u[:, 0:1], u[:, 1:2], u[:, 2:3]
    sh = jnp.concatenate([
        jnp.ones_like(ux), _s3 * ux, _s3 * uy, _s3 * uz,
        _s15 * ux * uy, _s15 * uy * uz, (_s5 / 2.0) * (3.0 * uz * uz - 1.0),
        _s15 * ux * uz, (_s15 / 2.0) * (ux * ux - uy * uy)], axis=1)
    f = jnp.concatenate(
        [h, sh, jnp.zeros((x.shape[0], D - 19), jnp.float32)], axis=1)
    qt, k, v = _qkv_from_f(f, Wq_ref[...], Wk_ref[...], Wv_ref[...],
                           Wd_ref[...], G_ref[...])
    qt_ref[...] = qt
    k_ref[...] = k
    v_ref[...] = v


def _call_embed(xp, eap, We, be, Wq, Wk, Wv, Wd, G):
    row = pl.BlockSpec((_RB, D), lambda i: (i, 0))
    full = pl.BlockSpec(None, lambda i: (0, 0))
    out = jax.ShapeDtypeStruct((NP, D), jnp.float32)
    return pl.pallas_call(
        _embed_body,
        grid=(NP // _RB,),
        in_specs=[pl.BlockSpec((_RB, 10), lambda i: (i, 0)),
                  pl.BlockSpec((_RB, 3), lambda i: (i, 0)),
                  full, full, full, full, full, full, full],
        out_specs=[row, row, row],
        out_shape=[out, out, out],
    )(xp, eap, We, be, Wq, Wk, Wv, Wd, G)


def _dense_body(has_prev, *refs):
    if has_prev:
        (agg_ref, vp_ref, fp_ref, Wq_ref, Wk_ref, Wv_ref, Wd_ref, G_ref,
         f_ref, qt_ref, k_ref, v_ref) = refs
        f = agg_ref[...] + vp_ref[...] + fp_ref[...]
    else:
        (agg_ref, vp_ref, Wq_ref, Wk_ref, Wv_ref, Wd_ref, G_ref,
         f_ref, qt_ref, k_ref, v_ref) = refs
        f = agg_ref[...] + vp_ref[...]
    qt, k, v = _qkv_from_f(f, Wq_ref[...], Wk_ref[...], Wv_ref[...],
                           Wd_ref[...], G_ref[...])
    f_ref[...] = f
    qt_ref[...] = qt
    k_ref[...] = k
    v_ref[...] = v


def _call_dense(agg, vprev, fprev, Wq, Wk, Wv, Wd, G):
    has_prev = fprev is not None
    row = pl.BlockSpec((_RB, D), lambda i: (i, 0))
    full = pl.BlockSpec(None, lambda i: (0, 0))
    out = jax.ShapeDtypeStruct((NP, D), jnp.float32)
    args = [agg, vprev] + ([fprev] if has_prev else []) + [Wq, Wk, Wv, Wd, G]
    n_row = 3 if has_prev else 2
    return pl.pallas_call(
        functools.partial(_dense_body, has_prev),
        grid=(NP // _RB,),
        in_specs=[row] * n_row + [full] * 5,
        out_specs=[row, row, row, row],
        out_shape=[out, out, out, out],
    )(*args)


def _head_body(agg_ref, vp_ref, fp_ref, degp_ref, batch_ref, Wol_ref,
               W1_ref, b1_ref, W2_ref, b2_ref, Wo_ref, bo_ref,
               out_ref, acc_s, acc_c):
    i = pl.program_id(0)

    @pl.when(i == 0)
    def _():
        acc_s[...] = jnp.zeros_like(acc_s)
        acc_c[...] = jnp.zeros_like(acc_c)

    f = agg_ref[...] + vp_ref[...] + fp_ref[...]
    s = _dot(f, Wol_ref[...])
    nrm = jnp.sqrt(jnp.sum(s * s, axis=1, keepdims=True))
    s = s / jnp.maximum(nrm, 1e-12)
    s = jax.nn.relu(_dot(s, W1_ref[...]) + b1_ref[...])
    s = jax.nn.relu(_dot(s, W2_ref[...]) + b2_ref[...])
    deg = (degp_ref[0, :] + degp_ref[1, :]).reshape(-1, 1)
    gids = lax.broadcasted_iota(jnp.int32, (1, NG), 1)
    onehot = (batch_ref[...] == gids).astype(jnp.float32)
    ws = s * deg
    acc_s[...] += lax.dot_general(onehot, ws, (((0,), (0,)), ((), ())),
                                  precision=_HP,
                                  preferred_element_type=jnp.float32)
    acc_c[...] += lax.dot_general(
        onehot, jnp.broadcast_to(deg, ws.shape), (((0,), (0,)), ((), ())),
        precision=_HP, preferred_element_type=jnp.float32)

    @pl.when(i == pl.num_programs(0) - 1)
    def _():
        m = acc_s[...] / jnp.maximum(acc_c[...], 1.0)
        logits = _dot(m, Wo_ref[...]) + bo_ref[...]
        mx = jnp.max(logits, axis=1, keepdims=True)
        e = jnp.exp(logits - mx)
        out_ref[...] = e / jnp.sum(e, axis=1, keepdims=True)


def _call_head(agg, vprev, fprev, degp, batch2d, Wol, W1, b1, W2, b2, Wo, bo):
    row = pl.BlockSpec((_RB, D), lambda i: (i, 0))
    full = pl.BlockSpec(None, lambda i: (0, 0))
    return pl.pallas_call(
        _head_body,
        grid=(NP // _RB,),
        in_specs=[row, row, row,
                  pl.BlockSpec((2, _RB), lambda i: (0, i)),
                  pl.BlockSpec((_RB, 1), lambda i: (i, 0)),
                  full, full, full, full, full, full, full],
        out_specs=pl.BlockSpec(None, lambda i: (0, 0)),
        out_shape=jax.ShapeDtypeStruct((NG, 9), jnp.float32),
        scratch_shapes=[pltpu.VMEM((NG, HID), jnp.float32),
                        pltpu.VMEM((NG, HID), jnp.float32)],
    )(agg, vprev, fprev, degp, batch2d, Wol, W1, b1, W2, b2, Wo, bo)


_MESH = dict(core_axis_name="c", subcore_axis_name="s",
             num_cores=NCORES, num_subcores=NSUB)


def _lane():
    return lax.broadcasted_iota(jnp.int32, (16,), 0)


def _scores_body(with_deg, *refs):
    if with_deg:
        (qt_hbm, k_hbm, dst_hbm, src_hbm, ex_hbm, zp_hbm, degp_hbm,
         dst_v, src_v, qrows, krows, ex_v, stage_v, ones_v,
         z_sp, deg_sp, semq0, semq1, semk0, semk1) = refs
    else:
        (qt_hbm, k_hbm, dst_hbm, src_hbm, ex_hbm, zp_hbm,
         dst_v, src_v, qrows, krows, ex_v, stage_v,
         z_sp, semq0, semq1, semk0, semk1) = refs
    semq = (semq0, semq1)
    semk = (semk0, semk1)
    c = lax.axis_index("c")
    s = lax.axis_index("s")
    wid = s * NCORES + c
    zseg = NP // NSUB
    sz = pl.multiple_of(s * zseg, zseg)
    lane = _lane()

    def zero_loop(i, carry):
        stage_v[pl.ds(pl.multiple_of(i * 16, 16), 16)] = jnp.zeros(
            (16,), jnp.float32)
        return carry
    lax.fori_loop(0, zseg // 16, zero_loop, 0)
    pltpu.sync_copy(stage_v, z_sp.at[pl.ds(sz, zseg)])
    if with_deg:
        pltpu.sync_copy(stage_v, deg_sp.at[pl.ds(sz, zseg)])
    plsc.subcore_barrier()

    nchunk = EP // 32 // CHUNK_A
    base0 = wid * (EP // 32)

    def start_chunk(ci, b):
        base = pl.multiple_of(base0 + ci * CHUNK_A, CHUNK_A)
        i1 = pltpu.async_copy(dst_hbm.at[pl.ds(base, CHUNK_A)],
                              dst_v.at[b], semq[b])
        i2 = pltpu.async_copy(src_hbm.at[pl.ds(base, CHUNK_A)],
                              src_v.at[b], semk[b])
        i1.wait()
        i2.wait()
        pltpu.async_copy(qt_hbm.at[dst_v.at[b]], qrows.at[b], semq[b])
        pltpu.async_copy(k_hbm.at[src_v.at[b]], krows.at[b], semk[b])

    def drain_chunk(b):
        pltpu.make_async_copy(qt_hbm.at[dst_v.at[b]], qrows.at[b],
                              semq[b]).wait()
        pltpu.make_async_copy(k_hbm.at[src_v.at[b]], krows.at[b],
                              semk[b]).wait()

    start_chunk(jnp.int32(0), 0)

    def pair_loop(pi, carry):
        for half in range(2):
            ci = pi * 2 + half
            b = half
            other = 1 - half
            drain_chunk(b)

            @pl.when(ci < nchunk - 1)
            def _():
                start_chunk(ci + 1, other)

            def grp_loop(g, carry2):
                o = pl.multiple_of(g * 16, 16)
                rows = o + lane
                acc = jnp.zeros((16,), jnp.float32)
                for j in range(DIM_ACT):
                    colj = jnp.full((16,), j, jnp.int32)
                    acc = acc + (plsc.load_gather(qrows.at[b],
                                                  [rows, colj]) *
                                 plsc.load_gather(krows.at[b],
                                                  [rows, colj]))
                ex_v[b, pl.ds(o, 16)] = jnp.exp(acc)
                if with_deg:
                    d16 = dst_v[b, pl.ds(o, 16)]
                    ones_v[b, pl.ds(o, 16)] = jnp.where(
                        d16 < N, jnp.float32(1.0), jnp.float32(0.0))
                return carry2
            lax.fori_loop(0, CHUNK_A // 16, grp_loop, 0)

            base = pl.multiple_of(base0 + ci * CHUNK_A, CHUNK_A)
            pltpu.sync_copy(ex_v.at[b], ex_hbm.at[pl.ds(base, CHUNK_A)])
            pltpu.sync_copy(ex_v.at[b], z_sp.at[dst_v.at[b]], add=True)
            if with_deg:
                pltpu.sync_copy(ones_v.at[b], deg_sp.at[dst_v.at[b]],
                                add=True)
        return carry
    lax.fori_loop(0, nchunk // 2, pair_loop, 0)
    plsc.subcore_barrier()

    pltpu.sync_copy(z_sp.at[pl.ds(sz, zseg)],
                    zp_hbm.at[c, pl.ds(sz, zseg)])
    if with_deg:
        pltpu.sync_copy(deg_sp.at[pl.ds(sz, zseg)],
                        degp_hbm.at[c, pl.ds(sz, zseg)])


def _make_scores(with_deg):
    out_type = [jax.ShapeDtypeStruct((EP,), jnp.float32),
                jax.ShapeDtypeStruct((NCORES, NP), jnp.float32)]
    scratch = [pltpu.VMEM((2, CHUNK_A), jnp.int32),
               pltpu.VMEM((2, CHUNK_A), jnp.int32),
               pltpu.VMEM((2, CHUNK_A, D), jnp.float32),
               pltpu.VMEM((2, CHUNK_A, D), jnp.float32),
               pltpu.VMEM((2, CHUNK_A), jnp.float32),
               pltpu.VMEM((NP // NSUB,), jnp.float32)]
    if with_deg:
        out_type.append(jax.ShapeDtypeStruct((NCORES, NP), jnp.float32))
        scratch.append(pltpu.VMEM((2, CHUNK_A), jnp.float32))
    scratch.append(pltpu.VMEM_SHARED((NP,), jnp.float32))
    if with_deg:
        scratch.append(pltpu.VMEM_SHARED((NP,), jnp.float32))
    scratch += [pltpu.SemaphoreType.DMA, pltpu.SemaphoreType.DMA,
                pltpu.SemaphoreType.DMA, pltpu.SemaphoreType.DMA]
    return pl.kernel(
        functools.partial(_scores_body, with_deg),
        out_type=tuple(out_type),
        mesh=plsc.VectorSubcoreMesh(**_MESH),
        scratch_types=scratch,
        compiler_params=pltpu.CompilerParams(needs_layout_passes=False, use_tc_tiling_on_sc=False),
    )


@functools.cache
def _scores_deg():
    return _make_scores(True)


@functools.cache
def _scores():
    return _make_scores(False)


def _rsqrt_pos(x):
    i = plsc.bitcast(x, jnp.int32)
    i = jnp.int32(0x5F3759DF) - lax.shift_right_logical(i, 1)
    y = plsc.bitcast(i, jnp.float32)
    for _ in range(3):
        y = y * (1.5 - 0.5 * x * y * y)
    return y


def _agg_body(v_hbm, ex_hbm, zp_hbm, dst_hbm, src_hbm, agg_hbm,
              dst_v, src_v, ex_v, st_src, st_loc, st_w, vrows, zloc, ztmp,
              fout_sp, sem1, semb0, semb1):
    c = lax.axis_index("c")
    s = lax.axis_index("s")
    lane = _lane()
    rows_per_tile = QUART // NSUB
    rbase = pl.multiple_of(s * rows_per_tile, rows_per_tile)
    nchunk = EP // NSUB // CHUNK_B
    base0 = s * (EP // NSUB)
    zero16f = jnp.zeros((16,), jnp.float32)
    zero16i = jnp.zeros((16,), jnp.int32)
    semb = (semb0, semb1)

    def start_scan(ci, b):
        base = pl.multiple_of(base0 + ci * CHUNK_B, CHUNK_B)
        pltpu.async_copy(dst_hbm.at[pl.ds(base, CHUNK_B)], dst_v.at[b],
                         semb[b])
        pltpu.async_copy(src_hbm.at[pl.ds(base, CHUNK_B)], src_v.at[b],
                         semb[b])
        pltpu.async_copy(ex_hbm.at[pl.ds(base, CHUNK_B)], ex_v.at[b],
                         semb[b])

    def drain_scan(ci, b):
        base = pl.multiple_of(base0 + ci * CHUNK_B, CHUNK_B)
        pltpu.make_async_copy(dst_hbm.at[pl.ds(base, CHUNK_B)], dst_v.at[b],
                              semb[b]).wait()
        pltpu.make_async_copy(src_hbm.at[pl.ds(base, CHUNK_B)], src_v.at[b],
                              semb[b]).wait()
        pltpu.make_async_copy(ex_hbm.at[pl.ds(base, CHUNK_B)], ex_v.at[b],
                              semb[b]).wait()

    def fire():
        pltpu.async_copy(v_hbm.at[st_src], vrows, sem1).wait()

        def scale_loop(g, carry):
            o = pl.multiple_of(g * 16, 16)
            rows = o + lane
            w16 = st_w[pl.ds(o, 16)]
            for j in range(DIM_ACT):
                colj = jnp.full((16,), j, jnp.int32)
                vc = plsc.load_gather(vrows, [rows, colj])
                plsc.store_scatter(vrows, [rows, colj], vc * w16)
            return carry
        lax.fori_loop(0, FIRE // 16, scale_loop, 0)
        pltpu.sync_copy(vrows, fout_sp.at[st_loc], add=True)

    def pad_tail(cur):
        def pad_loop(g, carry):
            idx = g * 16 + lane
            m = (idx >= cur) & (idx < FIRE)
            plsc.store_scatter(st_src, [idx], zero16i, mask=m)
            plsc.store_scatter(st_loc, [idx], zero16i, mask=m)
            plsc.store_scatter(st_w, [idx], zero16f, mask=m)
            return carry
        lax.fori_loop(0, FIRE // 16, pad_loop, 0)

    for pi in range(NPASS):
        lo = pl.multiple_of((pi * NCORES + c) * QUART, QUART)

        pltpu.sync_copy(zp_hbm.at[0, pl.ds(lo, QUART)], zloc)
        pltpu.sync_copy(zp_hbm.at[1, pl.ds(lo, QUART)], ztmp)

        def zadd(i, carry):
            o = pl.multiple_of(i * 16, 16)
            zloc[pl.ds(o, 16)] = zloc[pl.ds(o, 16)] + ztmp[pl.ds(o, 16)]
            return carry
        lax.fori_loop(0, QUART // 16, zadd, 0)

        def vz(i, carry):
            for cj in range(D // 16):
                vrows[i, pl.ds(cj * 16, 16)] = zero16f
            return carry
        lax.fori_loop(0, FIRE, vz, 0)
        done = 0
        while done < rows_per_tile:
            step = min(FIRE, rows_per_tile - done)
            pltpu.sync_copy(vrows.at[pl.ds(0, step)],
                            fout_sp.at[pl.ds(rbase + done, step)])
            done += step
        plsc.subcore_barrier()

        start_scan(jnp.int32(0), 0)

        def pair_loop(pi2, cur):
            for half in range(2):
                ci = pi2 * 2 + half
                b = half
                other = 1 - half
                drain_scan(ci, b)

                @pl.when(ci < nchunk - 1)
                def _():
                    start_scan(ci + 1, other)

                def grp_loop(g, cur2):
                    o = pl.multiple_of(g * 16, 16)
                    d16 = dst_v[b, pl.ds(o, 16)]
                    loc = d16 - lo
                    owned = (loc >= 0) & (loc < QUART)
                    locc = jnp.clip(loc, 0, QUART - 1)
                    zv = plsc.load_gather(zloc, [locc])
                    exv = ex_v[b, pl.ds(o, 16)]
                    ratio = jnp.where(
                        zv > 0.0, exv / jnp.where(zv > 0.0, zv, 1.0), 0.0)
                    w = jnp.where(ratio > 0.0, ratio * _rsqrt_pos(ratio),
                                  0.0)
                    ranks = plsc.cumsum(owned.astype(jnp.int32))
                    n = jnp.sum(owned.astype(jnp.int32), axis=0)
                    pos = jnp.clip(cur2 + ranks - 1, 0, FIRE - 1)
                    plsc.store_scatter(st_src, [pos],
                                       src_v[b, pl.ds(o, 16)], mask=owned)
                    plsc.store_scatter(st_loc, [pos], locc, mask=owned)
                    plsc.store_scatter(st_w, [pos], w, mask=owned)
                    cur3 = cur2 + n

                    @pl.when(cur3 > FIRE - 16)
                    def _():
                        pad_tail(cur3)
                        fire()
                    return jnp.where(cur3 > FIRE - 16, 0, cur3)
                cur = lax.fori_loop(0, GRP_B, grp_loop, cur)
            return cur
        cur = lax.fori_loop(0, nchunk // 2, pair_loop, jnp.int32(0))

        pad_tail(cur)
        fire()
        plsc.subcore_barrier()

        pltpu.sync_copy(fout_sp.at[pl.ds(rbase, rows_per_tile)],
                        agg_hbm.at[pl.ds(lo + rbase, rows_per_tile)])


@functools.cache
def _agg():
    return pl.kernel(
        _agg_body,
        out_type=jax.ShapeDtypeStruct((NP, D), jnp.float32),
        mesh=plsc.VectorSubcoreMesh(**_MESH),
        scratch_types=[pltpu.VMEM((2, CHUNK_B), jnp.int32),
                       pltpu.VMEM((2, CHUNK_B), jnp.int32),
                       pltpu.VMEM((2, CHUNK_B), jnp.float32),
                       pltpu.VMEM((FIRE,), jnp.int32),
                       pltpu.VMEM((FIRE,), jnp.int32),
                       pltpu.VMEM((FIRE,), jnp.float32),
                       pltpu.VMEM((FIRE, D), jnp.float32),
                       pltpu.VMEM((QUART,), jnp.float32),
                       pltpu.VMEM((QUART,), jnp.float32),
                       pltpu.VMEM_SHARED((QUART, D), jnp.float32),
                       pltpu.SemaphoreType.DMA, pltpu.SemaphoreType.DMA,
                       pltpu.SemaphoreType.DMA],
        compiler_params=pltpu.CompilerParams(
            needs_layout_passes=False, use_tc_tiling_on_sc=False),
    )


def kernel(x, edge_attr, params, edge_index, batch):
    p = params
    src = edge_index[0].astype(jnp.int32)
    dst = edge_index[1].astype(jnp.int32)

    xp = jnp.pad(x, ((0, NP - N), (0, 0)))
    eap = jnp.pad(edge_attr, ((0, NP - N), (0, 0)))
    padE = jnp.full((EP - E,), PAD_NODE, jnp.int32)
    dst_p = jnp.concatenate([dst, padE])
    src_p = jnp.concatenate([src, padE])
    batch2d = jnp.pad(batch.astype(jnp.int32), (0, NP - N)).reshape(NP, 1)

    G = _group_mat()
    Wq0, Wk0, Wv0, Wd0 = _layer_bigs(p['et'], (11, 1, 1))
    Wq1, Wk1, Wv1, Wd1 = _layer_bigs(p['m_et'][0], (11, 5, 5))
    Wq2, Wk2, Wv2, Wd2 = _layer_bigs(p['m_et'][1], (11, 5, 5))
    be = p['b_embd'].reshape(1, 10)
    Wol = jnp.zeros((D, HID), jnp.float32).at[:11, :].set(
        p['W_ol'] / np.sqrt(11.0))
    (W1h, b1h), (W2h, b2h) = p['lin']
    b1h = b1h.reshape(1, HID)
    b2h = b2h.reshape(1, HID)
    bo = p['b_out'].reshape(1, 9)

    qt, k, v = _call_embed(xp, eap, p['W_embd'], be, Wq0, Wk0, Wv0, Wd0, G)
    ex, zp, degp = _scores_deg()(qt, k, dst_p, src_p)
    agg = _agg()(v, ex, zp, dst_p, src_p)

    f1, qt, k, v1 = _call_dense(agg, v, None, Wq1, Wk1, Wv1, Wd1, G)
    ex, zp = _scores()(qt, k, dst_p, src_p)
    agg = _agg()(v1, ex, zp, dst_p, src_p)

    f2, qt, k, v2 = _call_dense(agg, v1, f1, Wq2, Wk2, Wv2, Wd2, G)
    ex, zp = _scores()(qt, k, dst_p, src_p)
    agg = _agg()(v2, ex, zp, dst_p, src_p)

    return _call_head(agg, v2, f2, degp, batch2d,
                      Wol, W1h, b1h, W2h, b2h, p['W_out'], bo)
